# baseline (speedup 1.0000x reference)
import dataclasses
import functools

import jax
import jax.numpy as jnp
from jax import lax
from jax.experimental import pallas as pl
from jax.experimental.pallas import tpu as pltpu
from jax.experimental.pallas import tpu_sc as plsc

_N = 10000
_NP = 10240
_E = 160000
_NB = 16
_NPB = _NP // _NB
_AR = 648
_NW = 32
_EPW = _E // _NW
_EPWP = 5008
_BR = 2000
_R = _N // _BR
_BMUL = 6554
_DH = 128
_C = 192


def _sc_mesh():
    return plsc.VectorSubcoreMesh(core_axis_name="c", subcore_axis_name="s")


def _sc_params():
    cp = pltpu.CompilerParams()
    if "needs_layout_passes" in pltpu.CompilerParams.__dataclass_fields__:
        cp = dataclasses.replace(cp, needs_layout_passes=False)
    return cp


def _lane_iota():
    return lax.iota(jnp.int32, 16)


def _masked_scalar(vec16, lane):
    return jnp.sum(jnp.where(_lane_iota() == lane, vec16, 0))


def _bucket_counts(dst):
    @functools.partial(
        pl.kernel,
        out_type=jax.ShapeDtypeStruct((_NW * _NB,), jnp.int32),
        mesh=_sc_mesh(),
        compiler_params=_sc_params(),
        scratch_types=[
            pltpu.VMEM((_EPWP,), jnp.int32),
            pltpu.VMEM((16,), jnp.int32),
        ],
    )
    def count_kernel(dst_hbm, out_hbm, dbuf, cvec):
        c = lax.axis_index("c")
        s = lax.axis_index("s")
        w = s * 2 + c
        pltpu.sync_copy(dst_hbm.at[pl.ds(w * _EPW, _EPW)],
                        dbuf.at[pl.ds(0, _EPW)])
        tail = dbuf[pl.ds(_EPWP - 16, 16)]
        dbuf[pl.ds(_EPWP - 16, 16)] = jnp.where(_lane_iota() < 8, tail, _NP)

        def body(j, cnt):
            d = dbuf[pl.ds(j * 16, 16)]
            b = (d * _BMUL) >> 22
            for bb in range(_NB):
                pop = plsc.all_reduce_population_count(b == bb)
                cnt = cnt + jnp.where(_lane_iota() == bb, pop, 0)
            return cnt

        cnt = lax.fori_loop(0, _EPWP // 16, body, jnp.zeros((16,), jnp.int32))
        cvec[...] = cnt
        pltpu.sync_copy(cvec, out_hbm.at[pl.ds(w * _NB, _NB)])

    return count_kernel(dst)


def _bucket_offsets(counts):
    @functools.partial(
        pl.kernel,
        out_type=[
            jax.ShapeDtypeStruct((_NW * _NB,), jnp.int32),
            jax.ShapeDtypeStruct((16,), jnp.int32),
        ],
        mesh=plsc.ScalarSubcoreMesh(axis_name="core", num_cores=2),
        scratch_types=[
            pltpu.SMEM((_NW * _NB,), jnp.int32),
            pltpu.SMEM((_NW * _NB,), jnp.int32),
            pltpu.SMEM((16,), jnp.int32),
            pltpu.SemaphoreType.DMA,
        ],
    )
    def offs_kernel(cnt_hbm, off_hbm, tot_hbm, cbuf, obuf, tbuf, sem):
        core = lax.axis_index("core")

        @pl.when(core == 0)
        def _():
            pltpu.async_copy(cnt_hbm, cbuf, sem).wait()

            @pl.loop(0, _NB)
            def _(b):
                def inner(w, run):
                    obuf[w * _NB + b] = b * _E + run
                    return run + cbuf[w * _NB + b]

                tbuf[b] = lax.fori_loop(0, _NW, inner, 0)

            pltpu.async_copy(obuf, off_hbm, sem).wait()
            pltpu.async_copy(tbuf, tot_hbm, sem).wait()

    return offs_kernel(counts)


def _bucket_scatter(src, dst, offs):
    @functools.partial(
        pl.kernel,
        out_type=jax.ShapeDtypeStruct((_NB * _E + 16,), jnp.int32),
        mesh=_sc_mesh(),
        compiler_params=_sc_params(),
        scratch_types=[
            pltpu.VMEM((_EPWP,), jnp.int32),
            pltpu.VMEM((_EPWP,), jnp.int32),
            pltpu.VMEM((_EPWP,), jnp.int32),
            pltpu.VMEM((_EPWP,), jnp.int32),
            pltpu.VMEM((16,), jnp.int32),
            pltpu.SMEM((16,), jnp.int32),
        ],
    )
    def scat_kernel(src_hbm, dst_hbm, off_hbm, out_hbm,
                    sbuf, dbuf, pbuf, pos, off_v, osm):
        c = lax.axis_index("c")
        s = lax.axis_index("s")
        w = s * 2 + c
        pltpu.sync_copy(src_hbm.at[pl.ds(w * _EPW, _EPW)],
                        sbuf.at[pl.ds(0, _EPW)])
        pltpu.sync_copy(dst_hbm.at[pl.ds(w * _EPW, _EPW)],
                        dbuf.at[pl.ds(0, _EPW)])
        pltpu.sync_copy(off_hbm.at[pl.ds(w * _NB, _NB)], off_v)
        ov = off_v[...]
        for bb in range(_NB):
            osm[bb] = _masked_scalar(ov, bb)
        tail = dbuf[pl.ds(_EPWP - 16, 16)]
        dbuf[pl.ds(_EPWP - 16, 16)] = jnp.where(_lane_iota() < 8, tail, _NP)

        @pl.loop(0, _EPWP // 16)
        def _(j):
            sl = pl.ds(j * 16, 16)
            pbuf[sl] = sbuf[sl] | (dbuf[sl] << 14)

        @pl.loop(0, _EPWP // 16)
        def _(j):
            sl = pl.ds(j * 16, 16)
            b = (dbuf[sl] * _BMUL) >> 22
            posv = _NB * _E + _lane_iota()
            for bb in range(_NB):
                m = b == bb
                mi = m.astype(jnp.int32)
                csum = jnp.cumsum(mi)
                obb = osm[bb]
                posv = jnp.where(m, csum - 1 + obb, posv)
                osm[bb] = obb + jnp.sum(mi)
            pos[sl] = posv

        pltpu.sync_copy(pbuf, out_hbm.at[pos])

    return scat_kernel(src, dst, offs)


def _degrees(bedges, tot, zeros16):
    C = 1920

    @functools.partial(
        pl.kernel,
        out_type=jax.ShapeDtypeStruct((_NP, 16), jnp.float32),
        mesh=_sc_mesh(),
        compiler_params=_sc_params(),
        scratch_types=[
            pltpu.VMEM((C,), jnp.int32),
            pltpu.VMEM((C,), jnp.int32),
            pltpu.VMEM((_AR, 16), jnp.float32),
            pltpu.VMEM((16,), jnp.int32),
        ],
    )
    def deg_kernel(be_hbm, tot_hbm, z_hbm, out_hbm, ebuf, dloc, acc, totv):
        c = lax.axis_index("c")
        t = lax.axis_index("s")

        @pl.when(c == 0)
        def _():
            pltpu.sync_copy(z_hbm, acc)
            pltpu.sync_copy(tot_hbm, totv)
            cnt = _masked_scalar(totv[...], t)
            nch = (cnt + (C - 1)) // C
            one = jnp.ones((16,), jnp.float32)
            cols = _lane_iota()

            def chunk(g, carry):
                pltpu.sync_copy(be_hbm.at[pl.ds(t * _E + g * C, C)], ebuf)
                rem = cnt - g * C

                @pl.loop(0, C // 16)
                def _(j):
                    sl = pl.ds(j * 16, 16)
                    v = ebuf[sl]
                    m = (j * 16 + _lane_iota()) < rem
                    dloc[sl] = jnp.where(m, (v >> 14) - t * _NPB, _NPB)

                def edge_body(e, cc):
                    es = jnp.full((16,), e, jnp.int32)
                    row = plsc.load_gather(dloc, [es])
                    plsc.addupdate_scatter(acc, [row, cols], one)
                    return cc

                lax.fori_loop(0, C, edge_body, 0)
                return carry

            lax.fori_loop(0, nch, chunk, 0)
            pltpu.sync_copy(acc.at[pl.ds(0, _NPB)],
                            out_hbm.at[pl.ds(t * _NPB, _NPB)])

    return deg_kernel(bedges, tot, zeros16)


def _edge_aggregate_spmem(hs, bedges, tot, zeros320):
    C = 800
    NH = _NP // 4
    AR2 = NH + 8

    @functools.partial(
        pl.kernel,
        out_type=jax.ShapeDtypeStruct((2, _NP, _DH), jnp.float32),
        mesh=_sc_mesh(),
        compiler_params=_sc_params(),
        scratch_types=[
            pltpu.VMEM((C,), jnp.int32),
            pltpu.VMEM((C,), jnp.int32),
            pltpu.VMEM((C,), jnp.int32),
            pltpu.VMEM((C, _DH), jnp.float32),
            pltpu.VMEM((16,), jnp.int32),
            pltpu.VMEM_SHARED((AR2, _DH), jnp.float32),
        ],
    )
    def agg_kernel(hs_hbm, be_hbm, tot_hbm, z_hbm, out_hbm,
                   ebuf, sidx, dloc, buf, totv, acc):
        c = lax.axis_index("c")
        t = lax.axis_index("s")
        pltpu.sync_copy(tot_hbm, totv)

        for h in (0, 1, 2, 3):
            b = h * 4 + t // 4
            sub = t % 4
            pltpu.sync_copy(z_hbm, acc.at[pl.ds(t * (NH // 16), NH // 16)])
            cnt = _masked_scalar(totv[...], b)
            qs = ((cnt + 3) // 4 + 7) // 8 * 8
            start = b * _E + sub * qs
            mycnt = jnp.maximum(
                0, jnp.minimum(qs, cnt - sub * qs))
            nch = (mycnt + (C - 1)) // C
            plsc.subcore_barrier()

            def chunk(g, carry):
                pltpu.sync_copy(be_hbm.at[pl.ds(start + g * C, C)], ebuf)
                rem = mycnt - g * C

                @pl.loop(0, C // 16)
                def _(j):
                    sl = pl.ds(j * 16, 16)
                    v = ebuf[sl]
                    m = (j * 16 + _lane_iota()) < rem
                    sidx[sl] = jnp.where(m, v & 16383, 0)
                    dloc[sl] = jnp.where(m, (v >> 14) - h * NH, NH)

                pltpu.sync_copy(hs_hbm.at[c].at[sidx], buf)
                pltpu.sync_copy(buf, acc.at[dloc], add=True)
                return carry

            lax.fori_loop(0, nch, chunk, 0)
            plsc.subcore_barrier()

            @pl.when(sub == 0)
            def _():
                loc0 = (t // 4) * _NPB
                pltpu.sync_copy(
                    acc.at[pl.ds(loc0, _NPB)],
                    out_hbm.at[c, pl.ds(h * NH + loc0, _NPB)])
            plsc.subcore_barrier()

    return agg_kernel(hs, bedges, tot, zeros320)


def _dinv_from_deg(deg):
    def body(d_ref, o_ref):
        o_ref[...] = lax.rsqrt(d_ref[...][:, 0:1] + 1.0)

    return pl.pallas_call(
        body, out_shape=jax.ShapeDtypeStruct((_NP, 1), jnp.float32))(deg)


def _pad128(h):
    d = h.shape[1]
    if d == _DH:
        return h
    return jnp.concatenate(
        [h, jnp.zeros((h.shape[0], _DH - d), jnp.float32)], axis=-1)


def _tc_first(x, W, dinv2d):
    din, d = W.shape
    dh = d // 2

    def body(x_ref, w_ref, dinv_ref, hs_ref):
        h = jnp.dot(x_ref[...], w_ref[...],
                    preferred_element_type=jnp.float32,
                    precision=lax.Precision.HIGHEST)
        hs = h * dinv_ref[...]
        hs_ref[0] = hs[:, :dh]
        hs_ref[1] = hs[:, dh:]

    return pl.pallas_call(
        body,
        grid=(_R,),
        in_specs=[
            pl.BlockSpec((_BR, din), lambda r: (r, 0)),
            pl.BlockSpec((din, d), lambda r: (0, 0)),
            pl.BlockSpec((_BR, 1), lambda r: (r, 0)),
        ],
        out_specs=pl.BlockSpec((2, _BR, dh), lambda r: (0, r, 0)),
        out_shape=jax.ShapeDtypeStruct((2, _N, dh), jnp.float32),
    )(x, W, dinv2d)


def _tc_layer(agg, hs, b2d, dinv2d, W, skip, flag2d):
    d = 256
    dh = _DH

    def body(agg_r, hs_r, b_r, dinv_r, w_r, skip_r, f_r, act_ref, hs_ref):
        aggv = agg_r[...]
        hsv = hs_r[...]
        pre = jnp.concatenate([aggv[0] + hsv[0], aggv[1] + hsv[1]], axis=-1)
        a = dinv_r[...] * pre + b_r[...] + f_r[...] * skip_r[...]
        act = jnp.maximum(a, 0.0)
        act_ref[...] = act
        h = jnp.dot(act, w_r[...], preferred_element_type=jnp.float32,
                    precision=lax.Precision.HIGHEST)
        hsn = h * dinv_r[...]
        hs_ref[0] = hsn[:, :dh]
        hs_ref[1] = hsn[:, dh:]

    return pl.pallas_call(
        body,
        grid=(_R,),
        in_specs=[
            pl.BlockSpec((2, _BR, _DH), lambda r: (0, r, 0)),
            pl.BlockSpec((2, _BR, _DH), lambda r: (0, r, 0)),
            pl.BlockSpec((1, d), lambda r: (0, 0)),
            pl.BlockSpec((_BR, 1), lambda r: (r, 0)),
            pl.BlockSpec((d, d), lambda r: (0, 0)),
            pl.BlockSpec((_BR, d), lambda r: (r, 0)),
            pl.BlockSpec((1, 1), lambda r: (0, 0)),
        ],
        out_specs=[
            pl.BlockSpec((_BR, d), lambda r: (r, 0)),
            pl.BlockSpec((2, _BR, dh), lambda r: (0, r, 0)),
        ],
        out_shape=[
            jax.ShapeDtypeStruct((_N, d), jnp.float32),
            jax.ShapeDtypeStruct((2, _N, dh), jnp.float32),
        ],
    )(agg, hs, b2d, dinv2d, W, skip, flag2d)


def _tc_head(act7p, Wp, bop):
    def body(a_r, w_r, b_r, z_ref):
        z_ref[...] = jnp.dot(a_r[...], w_r[...],
                             preferred_element_type=jnp.float32,
                             precision=lax.Precision.HIGHEST) + b_r[...]

    return pl.pallas_call(
        body,
        grid=(_R,),
        in_specs=[
            pl.BlockSpec((_BR, 256), lambda r: (r, 0)),
            pl.BlockSpec((256, 128), lambda r: (0, 0)),
            pl.BlockSpec((1, 128), lambda r: (0, 0)),
        ],
        out_specs=pl.BlockSpec((_BR, 128), lambda r: (r, 0)),
        out_shape=jax.ShapeDtypeStruct((_N, 128), jnp.float32),
    )(act7p, Wp, bop)


def kernel(x, edge_index, W1, b1, W2, b2, W3, b3, W4, b4, W5, b5, W6, b6,
           W7, b7, W_out, b_out):
    src = edge_index[0]
    dst = edge_index[1]
    zeros320 = jnp.zeros((_NP // 64, _DH), jnp.float32)
    zeros16 = jnp.zeros((_AR, 16), jnp.float32)

    counts = _bucket_counts(dst)
    offs, tot = _bucket_offsets(counts)
    bedges = _bucket_scatter(src, dst, offs)
    deg = _degrees(bedges, tot, zeros16)
    dinv2d = _dinv_from_deg(deg)

    def pad_w(W):
        return jnp.pad(W, ((0, 256 - W.shape[0]), (0, 256 - W.shape[1])))

    def pad_b(b):
        return jnp.pad(b, (0, 256 - b.shape[0]))

    Wstack = jnp.stack([pad_w(W) for W in (W2, W3, W4, W5, W6, W7)]
                       + [jnp.zeros((256, 256), jnp.float32)])
    bstack = jnp.stack([pad_b(b) for b in (b1, b2, b3, b4, b5, b6, b7)])
    flags = jnp.array([0., 1., 0., 1., 0., 1., 0.], jnp.float32)

    hs1 = _tc_first(x, W1, dinv2d)
    act0 = jnp.zeros((_N, 256), jnp.float32)

    def body(carry, xs):
        hs, act_prev = carry
        W, b, flag = xs
        agg = _edge_aggregate_spmem(hs, bedges, tot, zeros320)
        act, hs_next = _tc_layer(agg, hs, b.reshape(1, -1), dinv2d, W,
                                 act_prev, flag.reshape(1, 1))
        return (hs_next, act), None

    (_, act7p), _ = lax.scan(body, (hs1, act0), (Wstack, bstack, flags))

    Wp = jnp.pad(W_out, ((0, 256 - W_out.shape[0]), (0, 128 - W_out.shape[1])))
    bop = jnp.pad(b_out, (0, 128 - b_out.shape[0])).reshape(1, -1)
    z_pad = _tc_head(act7p, Wp, bop)
    return (act7p[:, : 32], z_pad[:, : b_out.shape[0]])


# speedup vs baseline: 5.1237x; 5.1237x over previous
import dataclasses
import functools

import jax
import jax.numpy as jnp
from jax import lax
from jax.experimental import pallas as pl
from jax.experimental.pallas import tpu as pltpu
from jax.experimental.pallas import tpu_sc as plsc

_N = 10000
_NP = 10240
_E = 160000
_NB = 16
_NPB = _NP // _NB
_AR = 648
_NW = 32
_EPW = _E // _NW
_EPWP = 5008
_BR = 2000
_R = _N // _BR
_BMUL = 6554
_DH = 128
_C = 192


def _sc_mesh():
    return plsc.VectorSubcoreMesh(core_axis_name="c", subcore_axis_name="s")


def _sc_params():
    cp = pltpu.CompilerParams()
    if "needs_layout_passes" in pltpu.CompilerParams.__dataclass_fields__:
        cp = dataclasses.replace(cp, needs_layout_passes=False)
    return cp


def _lane_iota():
    return lax.iota(jnp.int32, 16)


def _masked_scalar(vec16, lane):
    return jnp.sum(jnp.where(_lane_iota() == lane, vec16, 0))


def _bucket_counts(dst):
    @functools.partial(
        pl.kernel,
        out_type=jax.ShapeDtypeStruct((_NW * _NB,), jnp.int32),
        mesh=_sc_mesh(),
        compiler_params=_sc_params(),
        scratch_types=[
            pltpu.VMEM((_EPWP,), jnp.int32),
            pltpu.VMEM((16,), jnp.int32),
        ],
    )
    def count_kernel(dst_hbm, out_hbm, dbuf, cvec):
        c = lax.axis_index("c")
        s = lax.axis_index("s")
        w = s * 2 + c
        pltpu.sync_copy(dst_hbm.at[pl.ds(w * _EPW, _EPW)],
                        dbuf.at[pl.ds(0, _EPW)])
        tail = dbuf[pl.ds(_EPWP - 16, 16)]
        dbuf[pl.ds(_EPWP - 16, 16)] = jnp.where(_lane_iota() < 8, tail, _NP)

        def body(j, cnt):
            d = dbuf[pl.ds(j * 16, 16)]
            b = (d * _BMUL) >> 22
            for bb in range(_NB):
                pop = plsc.all_reduce_population_count(b == bb)
                cnt = cnt + jnp.where(_lane_iota() == bb, pop, 0)
            return cnt

        cnt = lax.fori_loop(0, _EPWP // 16, body, jnp.zeros((16,), jnp.int32))
        cvec[...] = cnt
        pltpu.sync_copy(cvec, out_hbm.at[pl.ds(w * _NB, _NB)])

    return count_kernel(dst)


def _bucket_offsets(counts):
    @functools.partial(
        pl.kernel,
        out_type=[
            jax.ShapeDtypeStruct((_NW * _NB,), jnp.int32),
            jax.ShapeDtypeStruct((16,), jnp.int32),
        ],
        mesh=plsc.ScalarSubcoreMesh(axis_name="core", num_cores=2),
        scratch_types=[
            pltpu.SMEM((_NW * _NB,), jnp.int32),
            pltpu.SMEM((_NW * _NB,), jnp.int32),
            pltpu.SMEM((16,), jnp.int32),
            pltpu.SemaphoreType.DMA,
        ],
    )
    def offs_kernel(cnt_hbm, off_hbm, tot_hbm, cbuf, obuf, tbuf, sem):
        core = lax.axis_index("core")

        @pl.when(core == 0)
        def _():
            pltpu.async_copy(cnt_hbm, cbuf, sem).wait()

            @pl.loop(0, _NB)
            def _(b):
                def inner(w, run):
                    obuf[w * _NB + b] = b * _E + run
                    return run + cbuf[w * _NB + b]

                tbuf[b] = lax.fori_loop(0, _NW, inner, 0)

            pltpu.async_copy(obuf, off_hbm, sem).wait()
            pltpu.async_copy(tbuf, tot_hbm, sem).wait()

    return offs_kernel(counts)


def _bucket_scatter(src, dst, offs):
    @functools.partial(
        pl.kernel,
        out_type=jax.ShapeDtypeStruct((_NB * _E + 16,), jnp.int32),
        mesh=_sc_mesh(),
        compiler_params=_sc_params(),
        scratch_types=[
            pltpu.VMEM((_EPWP,), jnp.int32),
            pltpu.VMEM((_EPWP,), jnp.int32),
            pltpu.VMEM((_EPWP,), jnp.int32),
            pltpu.VMEM((_EPWP,), jnp.int32),
            pltpu.VMEM((16,), jnp.int32),
            pltpu.SMEM((16,), jnp.int32),
        ],
    )
    def scat_kernel(src_hbm, dst_hbm, off_hbm, out_hbm,
                    sbuf, dbuf, pbuf, pos, off_v, osm):
        c = lax.axis_index("c")
        s = lax.axis_index("s")
        w = s * 2 + c
        pltpu.sync_copy(src_hbm.at[pl.ds(w * _EPW, _EPW)],
                        sbuf.at[pl.ds(0, _EPW)])
        pltpu.sync_copy(dst_hbm.at[pl.ds(w * _EPW, _EPW)],
                        dbuf.at[pl.ds(0, _EPW)])
        pltpu.sync_copy(off_hbm.at[pl.ds(w * _NB, _NB)], off_v)
        ov = off_v[...]
        for bb in range(_NB):
            osm[bb] = _masked_scalar(ov, bb)
        tail = dbuf[pl.ds(_EPWP - 16, 16)]
        dbuf[pl.ds(_EPWP - 16, 16)] = jnp.where(_lane_iota() < 8, tail, _NP)

        @pl.loop(0, _EPWP // 16)
        def _(j):
            sl = pl.ds(j * 16, 16)
            pbuf[sl] = sbuf[sl] | (dbuf[sl] << 14)

        @pl.loop(0, _EPWP // 16)
        def _(j):
            sl = pl.ds(j * 16, 16)
            b = (dbuf[sl] * _BMUL) >> 22
            posv = _NB * _E + _lane_iota()
            for bb in range(_NB):
                m = b == bb
                mi = m.astype(jnp.int32)
                csum = jnp.cumsum(mi)
                obb = osm[bb]
                posv = jnp.where(m, csum - 1 + obb, posv)
                osm[bb] = obb + jnp.sum(mi)
            pos[sl] = posv

        pltpu.sync_copy(pbuf, out_hbm.at[pos])

    return scat_kernel(src, dst, offs)


def _degrees(bedges, tot, zeros16):
    C = 1920

    @functools.partial(
        pl.kernel,
        out_type=jax.ShapeDtypeStruct((_NP, 16), jnp.float32),
        mesh=_sc_mesh(),
        compiler_params=_sc_params(),
        scratch_types=[
            pltpu.VMEM((C,), jnp.int32),
            pltpu.VMEM((C,), jnp.int32),
            pltpu.VMEM((_AR, 16), jnp.float32),
            pltpu.VMEM((16,), jnp.int32),
        ],
    )
    def deg_kernel(be_hbm, tot_hbm, z_hbm, out_hbm, ebuf, dloc, acc, totv):
        c = lax.axis_index("c")
        t = lax.axis_index("s")

        @pl.when(c == 0)
        def _():
            pltpu.sync_copy(z_hbm, acc)
            pltpu.sync_copy(tot_hbm, totv)
            cnt = _masked_scalar(totv[...], t)
            nch = (cnt + (C - 1)) // C
            one = jnp.ones((16,), jnp.float32)
            cols = _lane_iota()

            def chunk(g, carry):
                pltpu.sync_copy(be_hbm.at[pl.ds(t * _E + g * C, C)], ebuf)
                rem = cnt - g * C

                @pl.loop(0, C // 16)
                def _(j):
                    sl = pl.ds(j * 16, 16)
                    v = ebuf[sl]
                    m = (j * 16 + _lane_iota()) < rem
                    dloc[sl] = jnp.where(m, (v >> 14) - t * _NPB, _NPB)

                def edge_body(e, cc):
                    es = jnp.full((16,), e, jnp.int32)
                    row = plsc.load_gather(dloc, [es])
                    plsc.addupdate_scatter(acc, [row, cols], one)
                    return cc

                lax.fori_loop(0, C, edge_body, 0)
                return carry

            lax.fori_loop(0, nch, chunk, 0)
            pltpu.sync_copy(acc.at[pl.ds(0, _NPB)],
                            out_hbm.at[pl.ds(t * _NPB, _NPB)])

    return deg_kernel(bedges, tot, zeros16)


def _edge_aggregate(hs, bedges, tot, zeros, col_mode):
    SB = 3168
    C = 96

    @functools.partial(
        pl.kernel,
        out_type=jax.ShapeDtypeStruct((2, _NP, _DH), jnp.float32),
        mesh=_sc_mesh(),
        compiler_params=_sc_params(),
        scratch_types=[
            pltpu.VMEM((SB,), jnp.int32),
            pltpu.VMEM((SB,), jnp.int32),
            pltpu.VMEM((SB,), jnp.int32),
            pltpu.VMEM((C, _DH), jnp.float32),
            pltpu.VMEM((C, _DH), jnp.float32),
            pltpu.VMEM((_AR, _DH), jnp.float32),
            pltpu.VMEM((16,), jnp.int32),
            pltpu.SemaphoreType.DMA,
            pltpu.SemaphoreType.DMA,
        ],
    )
    def agg_kernel(hs_hbm, be_hbm, tot_hbm, z_hbm, out_hbm,
                   ebuf, sidx, dloc, buf0, buf1, acc, totv, sem0, sem1):
        c = lax.axis_index("c")
        t = lax.axis_index("s")
        pltpu.sync_copy(z_hbm, acc)
        pltpu.sync_copy(tot_hbm, totv)
        cnt = _masked_scalar(totv[...], t)
        if col_mode:
            start = t * _E
            mycnt = cnt
        else:
            half = ((cnt + 1) // 2 + 7) // 8 * 8
            start = t * _E + c * half
            mycnt = jnp.where(c == 0, jnp.minimum(half, cnt),
                              jnp.maximum(cnt - half, 0))
        nsb = (mycnt + (SB - 1)) // SB
        bufs = (buf0, buf1)
        sems = (sem0, sem1)

        def gather(g, par):
            return pltpu.make_async_copy(
                hs_hbm.at[c].at[sidx.at[pl.ds(g * C, C)]] if col_mode
                else hs_hbm.at[sidx.at[pl.ds(g * C, C)]],
                bufs[par], sems[par])

        def accumulate(base, bufP):
            def edge_body(e, cc):
                for u in range(2):
                    ee = 2 * e + u
                    es = jnp.full((16,), base + ee, jnp.int32)
                    row = plsc.load_gather(dloc, [es])
                    for cb in range(_DH // 16):
                        vals = bufP[ee, pl.ds(cb * 16, 16)]
                        plsc.addupdate_scatter(
                            acc, [row, cb * 16 + _lane_iota()], vals)
                return cc

            lax.fori_loop(0, C // 2, edge_body, 0)

        def super_chunk(sb, carry):
            sbase = start + sb * SB
            scnt = jnp.minimum(SB, mycnt - sb * SB)
            pltpu.sync_copy(be_hbm.at[pl.ds(sbase, SB)], ebuf)

            @pl.loop(0, SB // 16)
            def _(j):
                sl = pl.ds(j * 16, 16)
                v = ebuf[sl]
                m = (j * 16 + _lane_iota()) < scnt
                sidx[sl] = jnp.where(m, v & 16383, 0)
                dloc[sl] = jnp.where(m, (v >> 14) - t * _NPB, _NPB)

            nc2 = (scnt + (C - 1)) // C
            gather(0, 0).start()

            def pair(gp, cc):
                for par in (0, 1):
                    g = 2 * gp + par

                    @pl.when(g < nc2)
                    def _():
                        @pl.when(g + 1 < nc2)
                        def _():
                            gather(g + 1, 1 - par).start()

                        gather(g, par).wait()
                        accumulate(g * C, bufs[par])
                return cc

            lax.fori_loop(0, (nc2 + 1) // 2, pair, 0)
            return carry

        lax.fori_loop(0, nsb, super_chunk, 0)
        pltpu.sync_copy(acc.at[pl.ds(0, _NPB)],
                        out_hbm.at[c, pl.ds(t * _NPB, _NPB)])

    return agg_kernel(hs, bedges, tot, zeros)


def _dinv_from_deg(deg):
    def body(d_ref, o_ref):
        o_ref[...] = lax.rsqrt(d_ref[...][:, 0:1] + 1.0)

    return pl.pallas_call(
        body, out_shape=jax.ShapeDtypeStruct((_NP, 1), jnp.float32))(deg)


def _pad128(h):
    d = h.shape[1]
    if d == _DH:
        return h
    return jnp.concatenate(
        [h, jnp.zeros((h.shape[0], _DH - d), jnp.float32)], axis=-1)


def _tc_first(x, W, dinv2d):
    din, d = W.shape
    dh = d // 2

    def body(x_ref, w_ref, dinv_ref, hs_ref):
        h = jnp.dot(x_ref[...], w_ref[...],
                    preferred_element_type=jnp.float32,
                    precision=lax.Precision.HIGHEST)
        hs = h * dinv_ref[...]
        hs_ref[0] = hs[:, :dh]
        hs_ref[1] = hs[:, dh:]

    return pl.pallas_call(
        body,
        grid=(_R,),
        in_specs=[
            pl.BlockSpec((_BR, din), lambda r: (r, 0)),
            pl.BlockSpec((din, d), lambda r: (0, 0)),
            pl.BlockSpec((_BR, 1), lambda r: (r, 0)),
        ],
        out_specs=pl.BlockSpec((2, _BR, dh), lambda r: (0, r, 0)),
        out_shape=jax.ShapeDtypeStruct((2, _N, dh), jnp.float32),
    )(x, W, dinv2d)


def _tc_mid(agg, hs, b2d, dinv2d, W, in_col, skip=None, emit_act=False):
    d_prev = b2d.shape[1]
    d = W.shape[1]
    out_col = d == 256
    dh = d // 2

    def body(*refs):
        agg_r, hs_r, b_r, dinv_r, w_r = refs[:5]
        pos = 5
        skip_r = None
        if skip is not None:
            skip_r = refs[pos]
            pos += 1
        outs = refs[pos:]
        aggv = agg_r[...]
        hsv = hs_r[...]
        if in_col:
            pre = jnp.concatenate([aggv[0] + hsv[0], aggv[1] + hsv[1]],
                                  axis=-1)
        else:
            pre = (aggv[0] + aggv[1] + hsv)[:, :d_prev]
        a = dinv_r[...] * pre + b_r[...]
        if skip_r is not None:
            a = a + skip_r[...]
        act = jnp.maximum(a, 0.0)
        o = 0
        if emit_act:
            outs[o][...] = act
            o += 1
        h = jnp.dot(act, w_r[...], preferred_element_type=jnp.float32,
                    precision=lax.Precision.HIGHEST)
        hsn = h * dinv_r[...]
        if out_col:
            outs[o][0] = hsn[:, :dh]
            outs[o][1] = hsn[:, dh:]
        else:
            outs[o][...] = _pad128(hsn)

    in_specs = [
        pl.BlockSpec((2, _BR, _DH), lambda r: (0, r, 0)),
        pl.BlockSpec((2, _BR, _DH), lambda r: (0, r, 0)) if in_col
        else pl.BlockSpec((_BR, _DH), lambda r: (r, 0)),
        pl.BlockSpec((1, d_prev), lambda r: (0, 0)),
        pl.BlockSpec((_BR, 1), lambda r: (r, 0)),
        pl.BlockSpec((W.shape[0], d), lambda r: (0, 0)),
    ]
    args = [agg, hs, b2d, dinv2d, W]
    if skip is not None:
        in_specs.append(pl.BlockSpec((_BR, d_prev), lambda r: (r, 0)))
        args.append(skip)
    out_specs, out_shapes = [], []
    if emit_act:
        out_specs.append(pl.BlockSpec((_BR, d_prev), lambda r: (r, 0)))
        out_shapes.append(jax.ShapeDtypeStruct((_N, d_prev), jnp.float32))
    if out_col:
        out_specs.append(pl.BlockSpec((2, _BR, dh), lambda r: (0, r, 0)))
        out_shapes.append(jax.ShapeDtypeStruct((2, _N, dh), jnp.float32))
    else:
        out_specs.append(pl.BlockSpec((_BR, _DH), lambda r: (r, 0)))
        out_shapes.append(jax.ShapeDtypeStruct((_N, _DH), jnp.float32))

    res = pl.pallas_call(
        body, grid=(_R,), in_specs=in_specs,
        out_specs=out_specs, out_shape=out_shapes,
    )(*args)
    if emit_act:
        return res
    return res[0]


def _tc_final(agg, hs, b2d, dinv2d, Wp, bop):
    d_prev = b2d.shape[1]
    dp = Wp.shape[1]

    def body(agg_r, hs_r, b_r, dinv_r, w_r, bo_r, act_ref, z_ref):
        aggv = agg_r[...]
        pre = (aggv[0] + aggv[1] + hs_r[...])[:, :d_prev]
        act = jnp.maximum(dinv_r[...] * pre + b_r[...], 0.0)
        act_ref[...] = act
        z_ref[...] = jnp.dot(act, w_r[...],
                             preferred_element_type=jnp.float32,
                             precision=lax.Precision.HIGHEST) + bo_r[...]

    return pl.pallas_call(
        body,
        grid=(_R,),
        in_specs=[
            pl.BlockSpec((2, _BR, _DH), lambda r: (0, r, 0)),
            pl.BlockSpec((_BR, _DH), lambda r: (r, 0)),
            pl.BlockSpec((1, d_prev), lambda r: (0, 0)),
            pl.BlockSpec((_BR, 1), lambda r: (r, 0)),
            pl.BlockSpec((d_prev, dp), lambda r: (0, 0)),
            pl.BlockSpec((1, dp), lambda r: (0, 0)),
        ],
        out_specs=[
            pl.BlockSpec((_BR, d_prev), lambda r: (r, 0)),
            pl.BlockSpec((_BR, dp), lambda r: (r, 0)),
        ],
        out_shape=[
            jax.ShapeDtypeStruct((_N, d_prev), jnp.float32),
            jax.ShapeDtypeStruct((_N, dp), jnp.float32),
        ],
    )(agg, hs, b2d, dinv2d, Wp, bop)


def kernel(x, edge_index, W1, b1, W2, b2, W3, b3, W4, b4, W5, b5, W6, b6,
           W7, b7, W_out, b_out):
    src = edge_index[0]
    dst = edge_index[1]
    zeros128 = jnp.zeros((_AR, _DH), jnp.float32)
    zeros16 = jnp.zeros((_AR, 16), jnp.float32)

    counts = _bucket_counts(dst)
    offs, tot = _bucket_offsets(counts)
    bedges = _bucket_scatter(src, dst, offs)
    deg = _degrees(bedges, tot, zeros16)
    dinv2d = _dinv_from_deg(deg)

    def agg_of(hs, col_mode):
        return _edge_aggregate(hs, bedges, tot, zeros128, col_mode)

    b2ds = [b.reshape(1, -1) for b in (b1, b2, b3, b4, b5, b6, b7)]

    hs1 = _tc_first(x, W1, dinv2d)
    agg1 = agg_of(hs1, True)
    act1, hs2 = _tc_mid(agg1, hs1, b2ds[0], dinv2d, W2, in_col=True,
                        emit_act=True)
    agg2 = agg_of(hs2, True)
    hs3 = _tc_mid(agg2, hs2, b2ds[1], dinv2d, W3, in_col=True, skip=act1)
    agg3 = agg_of(hs3, False)
    act3, hs4 = _tc_mid(agg3, hs3, b2ds[2], dinv2d, W4, in_col=False,
                        emit_act=True)
    agg4 = agg_of(hs4, False)
    hs5 = _tc_mid(agg4, hs4, b2ds[3], dinv2d, W5, in_col=False, skip=act3)
    agg5 = agg_of(hs5, False)
    act5, hs6 = _tc_mid(agg5, hs5, b2ds[4], dinv2d, W6, in_col=False,
                        emit_act=True)
    agg6 = agg_of(hs6, False)
    hs7 = _tc_mid(agg6, hs6, b2ds[5], dinv2d, W7, in_col=False, skip=act5)
    agg7 = agg_of(hs7, False)

    Wp = jnp.pad(W_out, ((0, 0), (0, 128 - W_out.shape[1])))
    bop = jnp.pad(b_out, (0, 128 - b_out.shape[0])).reshape(1, -1)
    h7, z_pad = _tc_final(agg7, hs7, b2ds[6], dinv2d, Wp, bop)
    return (h7, z_pad[:, : b_out.shape[0]])


# speedup vs baseline: 5.1281x; 1.0009x over previous
import dataclasses
import functools

import jax
import jax.numpy as jnp
from jax import lax
from jax.experimental import pallas as pl
from jax.experimental.pallas import tpu as pltpu
from jax.experimental.pallas import tpu_sc as plsc

_N = 10000
_NP = 10240
_E = 160000
_NB = 16
_NPB = _NP // _NB
_AR = 648
_NW = 32
_EPW = _E // _NW
_EPWP = 5008
_BR = 2000
_R = _N // _BR
_BMUL = 6554
_DH = 128
_C = 192


def _sc_mesh():
    return plsc.VectorSubcoreMesh(core_axis_name="c", subcore_axis_name="s")


def _sc_params():
    cp = pltpu.CompilerParams()
    if "needs_layout_passes" in pltpu.CompilerParams.__dataclass_fields__:
        cp = dataclasses.replace(cp, needs_layout_passes=False)
    return cp


def _lane_iota():
    return lax.iota(jnp.int32, 16)


def _masked_scalar(vec16, lane):
    return jnp.sum(jnp.where(_lane_iota() == lane, vec16, 0))


def _bucket_counts(dst):
    @functools.partial(
        pl.kernel,
        out_type=jax.ShapeDtypeStruct((_NW * _NB,), jnp.int32),
        mesh=_sc_mesh(),
        compiler_params=_sc_params(),
        scratch_types=[
            pltpu.VMEM((_EPWP,), jnp.int32),
            pltpu.VMEM((16,), jnp.int32),
        ],
    )
    def count_kernel(dst_hbm, out_hbm, dbuf, cvec):
        c = lax.axis_index("c")
        s = lax.axis_index("s")
        w = s * 2 + c
        pltpu.sync_copy(dst_hbm.at[pl.ds(w * _EPW, _EPW)],
                        dbuf.at[pl.ds(0, _EPW)])
        tail = dbuf[pl.ds(_EPWP - 16, 16)]
        dbuf[pl.ds(_EPWP - 16, 16)] = jnp.where(_lane_iota() < 8, tail, _NP)

        def body(j, cnt):
            d = dbuf[pl.ds(j * 16, 16)]
            b = (d * _BMUL) >> 22
            for bb in range(_NB):
                pop = plsc.all_reduce_population_count(b == bb)
                cnt = cnt + jnp.where(_lane_iota() == bb, pop, 0)
            return cnt

        cnt = lax.fori_loop(0, _EPWP // 16, body, jnp.zeros((16,), jnp.int32))
        cvec[...] = cnt
        pltpu.sync_copy(cvec, out_hbm.at[pl.ds(w * _NB, _NB)])

    return count_kernel(dst)


def _bucket_offsets(counts):
    @functools.partial(
        pl.kernel,
        out_type=[
            jax.ShapeDtypeStruct((_NW * _NB,), jnp.int32),
            jax.ShapeDtypeStruct((16,), jnp.int32),
        ],
        mesh=plsc.ScalarSubcoreMesh(axis_name="core", num_cores=2),
        scratch_types=[
            pltpu.SMEM((_NW * _NB,), jnp.int32),
            pltpu.SMEM((_NW * _NB,), jnp.int32),
            pltpu.SMEM((16,), jnp.int32),
            pltpu.SemaphoreType.DMA,
        ],
    )
    def offs_kernel(cnt_hbm, off_hbm, tot_hbm, cbuf, obuf, tbuf, sem):
        core = lax.axis_index("core")

        @pl.when(core == 0)
        def _():
            pltpu.async_copy(cnt_hbm, cbuf, sem).wait()

            @pl.loop(0, _NB)
            def _(b):
                def inner(w, run):
                    obuf[w * _NB + b] = b * _E + run
                    return run + cbuf[w * _NB + b]

                tbuf[b] = lax.fori_loop(0, _NW, inner, 0)

            pltpu.async_copy(obuf, off_hbm, sem).wait()
            pltpu.async_copy(tbuf, tot_hbm, sem).wait()

    return offs_kernel(counts)


def _bucket_scatter(src, dst, offs):
    @functools.partial(
        pl.kernel,
        out_type=jax.ShapeDtypeStruct((_NB * _E + 16,), jnp.int32),
        mesh=_sc_mesh(),
        compiler_params=_sc_params(),
        scratch_types=[
            pltpu.VMEM((_EPWP,), jnp.int32),
            pltpu.VMEM((_EPWP,), jnp.int32),
            pltpu.VMEM((_EPWP,), jnp.int32),
            pltpu.VMEM((_EPWP,), jnp.int32),
            pltpu.VMEM((16,), jnp.int32),
            pltpu.SMEM((16,), jnp.int32),
        ],
    )
    def scat_kernel(src_hbm, dst_hbm, off_hbm, out_hbm,
                    sbuf, dbuf, pbuf, pos, off_v, osm):
        c = lax.axis_index("c")
        s = lax.axis_index("s")
        w = s * 2 + c
        pltpu.sync_copy(src_hbm.at[pl.ds(w * _EPW, _EPW)],
                        sbuf.at[pl.ds(0, _EPW)])
        pltpu.sync_copy(dst_hbm.at[pl.ds(w * _EPW, _EPW)],
                        dbuf.at[pl.ds(0, _EPW)])
        pltpu.sync_copy(off_hbm.at[pl.ds(w * _NB, _NB)], off_v)
        ov = off_v[...]
        for bb in range(_NB):
            osm[bb] = _masked_scalar(ov, bb)
        tail = dbuf[pl.ds(_EPWP - 16, 16)]
        dbuf[pl.ds(_EPWP - 16, 16)] = jnp.where(_lane_iota() < 8, tail, _NP)

        @pl.loop(0, _EPWP // 16)
        def _(j):
            sl = pl.ds(j * 16, 16)
            pbuf[sl] = sbuf[sl] | (dbuf[sl] << 14)

        @pl.loop(0, _EPWP // 16)
        def _(j):
            sl = pl.ds(j * 16, 16)
            b = (dbuf[sl] * _BMUL) >> 22
            posv = _NB * _E + _lane_iota()
            for bb in range(_NB):
                m = b == bb
                mi = m.astype(jnp.int32)
                csum = jnp.cumsum(mi)
                obb = osm[bb]
                posv = jnp.where(m, csum - 1 + obb, posv)
                osm[bb] = obb + jnp.sum(mi)
            pos[sl] = posv

        pltpu.sync_copy(pbuf, out_hbm.at[pos])

    return scat_kernel(src, dst, offs)


def _degrees(bedges, tot, zeros16):
    C = 1920

    @functools.partial(
        pl.kernel,
        out_type=jax.ShapeDtypeStruct((2, _NP, 16), jnp.float32),
        mesh=_sc_mesh(),
        compiler_params=_sc_params(),
        scratch_types=[
            pltpu.VMEM((C,), jnp.int32),
            pltpu.VMEM((C,), jnp.int32),
            pltpu.VMEM((_AR, 16), jnp.float32),
            pltpu.VMEM((16,), jnp.int32),
        ],
    )
    def deg_kernel(be_hbm, tot_hbm, z_hbm, out_hbm, ebuf, dloc, acc, totv):
        c = lax.axis_index("c")
        t = lax.axis_index("s")
        pltpu.sync_copy(z_hbm, acc)
        pltpu.sync_copy(tot_hbm, totv)
        cnt = _masked_scalar(totv[...], t)
        half = ((cnt + 1) // 2 + 7) // 8 * 8
        start = t * _E + c * half
        mycnt = jnp.where(c == 0, jnp.minimum(half, cnt),
                          jnp.maximum(cnt - half, 0))
        nch = (mycnt + (C - 1)) // C
        one = jnp.ones((16,), jnp.float32)
        cols = _lane_iota()

        def chunk(g, carry):
            pltpu.sync_copy(be_hbm.at[pl.ds(start + g * C, C)], ebuf)
            rem = mycnt - g * C

            @pl.loop(0, C // 16)
            def _(j):
                sl = pl.ds(j * 16, 16)
                v = ebuf[sl]
                m = (j * 16 + _lane_iota()) < rem
                dloc[sl] = jnp.where(m, (v >> 14) - t * _NPB, _NPB)

            def edge_body(e, cc):
                for u in range(4):
                    es = jnp.full((16,), 4 * e + u, jnp.int32)
                    row = plsc.load_gather(dloc, [es])
                    plsc.addupdate_scatter(acc, [row, cols], one)
                return cc

            lax.fori_loop(0, C // 4, edge_body, 0)
            return carry

        lax.fori_loop(0, nch, chunk, 0)
        pltpu.sync_copy(acc.at[pl.ds(0, _NPB)],
                        out_hbm.at[c, pl.ds(t * _NPB, _NPB)])

    return deg_kernel(bedges, tot, zeros16)


def _edge_aggregate(hs, bedges, tot, zeros, col_mode):
    SB = 3168
    C = 96

    @functools.partial(
        pl.kernel,
        out_type=jax.ShapeDtypeStruct((2, _NP, _DH), jnp.float32),
        mesh=_sc_mesh(),
        compiler_params=_sc_params(),
        scratch_types=[
            pltpu.VMEM((SB,), jnp.int32),
            pltpu.VMEM((SB,), jnp.int32),
            pltpu.VMEM((SB,), jnp.int32),
            pltpu.VMEM((C, _DH), jnp.float32),
            pltpu.VMEM((C, _DH), jnp.float32),
            pltpu.VMEM((_AR, _DH), jnp.float32),
            pltpu.VMEM((16,), jnp.int32),
            pltpu.SemaphoreType.DMA,
            pltpu.SemaphoreType.DMA,
        ],
    )
    def agg_kernel(hs_hbm, be_hbm, tot_hbm, z_hbm, out_hbm,
                   ebuf, sidx, dloc, buf0, buf1, acc, totv, sem0, sem1):
        c = lax.axis_index("c")
        t = lax.axis_index("s")
        pltpu.sync_copy(z_hbm, acc)
        pltpu.sync_copy(tot_hbm, totv)
        cnt = _masked_scalar(totv[...], t)
        if col_mode:
            start = t * _E
            mycnt = cnt
        else:
            half = ((cnt + 1) // 2 + 7) // 8 * 8
            start = t * _E + c * half
            mycnt = jnp.where(c == 0, jnp.minimum(half, cnt),
                              jnp.maximum(cnt - half, 0))
        nsb = (mycnt + (SB - 1)) // SB
        bufs = (buf0, buf1)
        sems = (sem0, sem1)

        def gather(g, par):
            return pltpu.make_async_copy(
                hs_hbm.at[c].at[sidx.at[pl.ds(g * C, C)]] if col_mode
                else hs_hbm.at[sidx.at[pl.ds(g * C, C)]],
                bufs[par], sems[par])

        def accumulate(base, bufP):
            def edge_body(e, cc):
                for u in range(4):
                    ee = 4 * e + u
                    es = jnp.full((16,), base + ee, jnp.int32)
                    row = plsc.load_gather(dloc, [es])
                    for cb in range(_DH // 16):
                        vals = bufP[ee, pl.ds(cb * 16, 16)]
                        plsc.addupdate_scatter(
                            acc, [row, cb * 16 + _lane_iota()], vals)
                return cc

            lax.fori_loop(0, C // 4, edge_body, 0)

        def super_chunk(sb, carry):
            sbase = start + sb * SB
            scnt = jnp.minimum(SB, mycnt - sb * SB)
            pltpu.sync_copy(be_hbm.at[pl.ds(sbase, SB)], ebuf)

            @pl.loop(0, SB // 16)
            def _(j):
                sl = pl.ds(j * 16, 16)
                v = ebuf[sl]
                m = (j * 16 + _lane_iota()) < scnt
                sidx[sl] = jnp.where(m, v & 16383, 0)
                dloc[sl] = jnp.where(m, (v >> 14) - t * _NPB, _NPB)

            nc2 = (scnt + (C - 1)) // C
            gather(0, 0).start()

            def pair(gp, cc):
                for par in (0, 1):
                    g = 2 * gp + par

                    @pl.when(g < nc2)
                    def _():
                        @pl.when(g + 1 < nc2)
                        def _():
                            gather(g + 1, 1 - par).start()

                        gather(g, par).wait()
                        accumulate(g * C, bufs[par])
                return cc

            lax.fori_loop(0, (nc2 + 1) // 2, pair, 0)
            return carry

        lax.fori_loop(0, nsb, super_chunk, 0)
        pltpu.sync_copy(acc.at[pl.ds(0, _NPB)],
                        out_hbm.at[c, pl.ds(t * _NPB, _NPB)])

    return agg_kernel(hs, bedges, tot, zeros)


def _dinv_from_deg(deg):
    def body(d_ref, o_ref):
        v = d_ref[...]
        o_ref[...] = lax.rsqrt(v[0, :, 0:1] + v[1, :, 0:1] + 1.0)

    return pl.pallas_call(
        body, out_shape=jax.ShapeDtypeStruct((_NP, 1), jnp.float32))(deg)


def _pad128(h):
    d = h.shape[1]
    if d == _DH:
        return h
    return jnp.concatenate(
        [h, jnp.zeros((h.shape[0], _DH - d), jnp.float32)], axis=-1)


def _tc_first(x, W, dinv2d):
    din, d = W.shape
    dh = d // 2

    def body(x_ref, w_ref, dinv_ref, hs_ref):
        h = jnp.dot(x_ref[...], w_ref[...],
                    preferred_element_type=jnp.float32,
                    precision=lax.Precision.HIGHEST)
        hs = h * dinv_ref[...]
        hs_ref[0] = hs[:, :dh]
        hs_ref[1] = hs[:, dh:]

    return pl.pallas_call(
        body,
        grid=(_R,),
        in_specs=[
            pl.BlockSpec((_BR, din), lambda r: (r, 0)),
            pl.BlockSpec((din, d), lambda r: (0, 0)),
            pl.BlockSpec((_BR, 1), lambda r: (r, 0)),
        ],
        out_specs=pl.BlockSpec((2, _BR, dh), lambda r: (0, r, 0)),
        out_shape=jax.ShapeDtypeStruct((2, _N, dh), jnp.float32),
    )(x, W, dinv2d)


def _tc_mid(agg, hs, b2d, dinv2d, W, in_col, skip=None, emit_act=False):
    d_prev = b2d.shape[1]
    d = W.shape[1]
    out_col = d == 256
    dh = d // 2

    def body(*refs):
        agg_r, hs_r, b_r, dinv_r, w_r = refs[:5]
        pos = 5
        skip_r = None
        if skip is not None:
            skip_r = refs[pos]
            pos += 1
        outs = refs[pos:]
        aggv = agg_r[...]
        hsv = hs_r[...]
        if in_col:
            pre = jnp.concatenate([aggv[0] + hsv[0], aggv[1] + hsv[1]],
                                  axis=-1)
        else:
            pre = (aggv[0] + aggv[1] + hsv)[:, :d_prev]
        a = dinv_r[...] * pre + b_r[...]
        if skip_r is not None:
            a = a + skip_r[...]
        act = jnp.maximum(a, 0.0)
        o = 0
        if emit_act:
            outs[o][...] = act
            o += 1
        h = jnp.dot(act, w_r[...], preferred_element_type=jnp.float32,
                    precision=lax.Precision.HIGHEST)
        hsn = h * dinv_r[...]
        if out_col:
            outs[o][0] = hsn[:, :dh]
            outs[o][1] = hsn[:, dh:]
        else:
            outs[o][...] = _pad128(hsn)

    in_specs = [
        pl.BlockSpec((2, _BR, _DH), lambda r: (0, r, 0)),
        pl.BlockSpec((2, _BR, _DH), lambda r: (0, r, 0)) if in_col
        else pl.BlockSpec((_BR, _DH), lambda r: (r, 0)),
        pl.BlockSpec((1, d_prev), lambda r: (0, 0)),
        pl.BlockSpec((_BR, 1), lambda r: (r, 0)),
        pl.BlockSpec((W.shape[0], d), lambda r: (0, 0)),
    ]
    args = [agg, hs, b2d, dinv2d, W]
    if skip is not None:
        in_specs.append(pl.BlockSpec((_BR, d_prev), lambda r: (r, 0)))
        args.append(skip)
    out_specs, out_shapes = [], []
    if emit_act:
        out_specs.append(pl.BlockSpec((_BR, d_prev), lambda r: (r, 0)))
        out_shapes.append(jax.ShapeDtypeStruct((_N, d_prev), jnp.float32))
    if out_col:
        out_specs.append(pl.BlockSpec((2, _BR, dh), lambda r: (0, r, 0)))
        out_shapes.append(jax.ShapeDtypeStruct((2, _N, dh), jnp.float32))
    else:
        out_specs.append(pl.BlockSpec((_BR, _DH), lambda r: (r, 0)))
        out_shapes.append(jax.ShapeDtypeStruct((_N, _DH), jnp.float32))

    res = pl.pallas_call(
        body, grid=(_R,), in_specs=in_specs,
        out_specs=out_specs, out_shape=out_shapes,
    )(*args)
    if emit_act:
        return res
    return res[0]


def _tc_final(agg, hs, b2d, dinv2d, Wp, bop):
    d_prev = b2d.shape[1]
    dp = Wp.shape[1]

    def body(agg_r, hs_r, b_r, dinv_r, w_r, bo_r, act_ref, z_ref):
        aggv = agg_r[...]
        pre = (aggv[0] + aggv[1] + hs_r[...])[:, :d_prev]
        act = jnp.maximum(dinv_r[...] * pre + b_r[...], 0.0)
        act_ref[...] = act
        z_ref[...] = jnp.dot(act, w_r[...],
                             preferred_element_type=jnp.float32,
                             precision=lax.Precision.HIGHEST) + bo_r[...]

    return pl.pallas_call(
        body,
        grid=(_R,),
        in_specs=[
            pl.BlockSpec((2, _BR, _DH), lambda r: (0, r, 0)),
            pl.BlockSpec((_BR, _DH), lambda r: (r, 0)),
            pl.BlockSpec((1, d_prev), lambda r: (0, 0)),
            pl.BlockSpec((_BR, 1), lambda r: (r, 0)),
            pl.BlockSpec((d_prev, dp), lambda r: (0, 0)),
            pl.BlockSpec((1, dp), lambda r: (0, 0)),
        ],
        out_specs=[
            pl.BlockSpec((_BR, d_prev), lambda r: (r, 0)),
            pl.BlockSpec((_BR, dp), lambda r: (r, 0)),
        ],
        out_shape=[
            jax.ShapeDtypeStruct((_N, d_prev), jnp.float32),
            jax.ShapeDtypeStruct((_N, dp), jnp.float32),
        ],
    )(agg, hs, b2d, dinv2d, Wp, bop)


def kernel(x, edge_index, W1, b1, W2, b2, W3, b3, W4, b4, W5, b5, W6, b6,
           W7, b7, W_out, b_out):
    src = edge_index[0]
    dst = edge_index[1]
    zeros128 = jnp.zeros((_AR, _DH), jnp.float32)
    zeros16 = jnp.zeros((_AR, 16), jnp.float32)

    counts = _bucket_counts(dst)
    offs, tot = _bucket_offsets(counts)
    bedges = _bucket_scatter(src, dst, offs)
    deg = _degrees(bedges, tot, zeros16)
    dinv2d = _dinv_from_deg(deg)

    def agg_of(hs, col_mode):
        return _edge_aggregate(hs, bedges, tot, zeros128, col_mode)

    b2ds = [b.reshape(1, -1) for b in (b1, b2, b3, b4, b5, b6, b7)]

    hs1 = _tc_first(x, W1, dinv2d)
    agg1 = agg_of(hs1, True)
    act1, hs2 = _tc_mid(agg1, hs1, b2ds[0], dinv2d, W2, in_col=True,
                        emit_act=True)
    agg2 = agg_of(hs2, True)
    hs3 = _tc_mid(agg2, hs2, b2ds[1], dinv2d, W3, in_col=True, skip=act1)
    agg3 = agg_of(hs3, False)
    act3, hs4 = _tc_mid(agg3, hs3, b2ds[2], dinv2d, W4, in_col=False,
                        emit_act=True)
    agg4 = agg_of(hs4, False)
    hs5 = _tc_mid(agg4, hs4, b2ds[3], dinv2d, W5, in_col=False, skip=act3)
    agg5 = agg_of(hs5, False)
    act5, hs6 = _tc_mid(agg5, hs5, b2ds[4], dinv2d, W6, in_col=False,
                        emit_act=True)
    agg6 = agg_of(hs6, False)
    hs7 = _tc_mid(agg6, hs6, b2ds[5], dinv2d, W7, in_col=False, skip=act5)
    agg7 = agg_of(hs7, False)

    Wp = jnp.pad(W_out, ((0, 0), (0, 128 - W_out.shape[1])))
    bop = jnp.pad(b_out, (0, 128 - b_out.shape[0])).reshape(1, -1)
    h7, z_pad = _tc_final(agg7, hs7, b2ds[6], dinv2d, Wp, bop)
    return (h7, z_pad[:, : b_out.shape[0]])


# speedup vs baseline: 5.6645x; 1.1046x over previous
import dataclasses
import functools

import jax
import jax.numpy as jnp
from jax import lax
from jax.experimental import pallas as pl
from jax.experimental.pallas import tpu as pltpu
from jax.experimental.pallas import tpu_sc as plsc

_N = 10000
_NP = 10240
_E = 160000
_NB = 16
_NPB = _NP // _NB
_AR = 648
_NW = 32
_EPW = _E // _NW
_EPWP = 5008
_BR = 2000
_R = _N // _BR
_BMUL = 6554
_DH = 128
_C = 192


def _sc_mesh():
    return plsc.VectorSubcoreMesh(core_axis_name="c", subcore_axis_name="s")


def _sc_params():
    cp = pltpu.CompilerParams()
    if "needs_layout_passes" in pltpu.CompilerParams.__dataclass_fields__:
        cp = dataclasses.replace(cp, needs_layout_passes=False)
    return cp


def _lane_iota():
    return lax.iota(jnp.int32, 16)


def _masked_scalar(vec16, lane):
    return jnp.sum(jnp.where(_lane_iota() == lane, vec16, 0))


def _bucket_counts(dst):
    @functools.partial(
        pl.kernel,
        out_type=jax.ShapeDtypeStruct((_NW * _NB,), jnp.int32),
        mesh=_sc_mesh(),
        compiler_params=_sc_params(),
        scratch_types=[
            pltpu.VMEM((_EPWP,), jnp.int32),
            pltpu.VMEM((16,), jnp.int32),
        ],
    )
    def count_kernel(dst_hbm, out_hbm, dbuf, cvec):
        c = lax.axis_index("c")
        s = lax.axis_index("s")
        w = s * 2 + c
        pltpu.sync_copy(dst_hbm.at[pl.ds(w * _EPW, _EPW)],
                        dbuf.at[pl.ds(0, _EPW)])
        tail = dbuf[pl.ds(_EPWP - 16, 16)]
        dbuf[pl.ds(_EPWP - 16, 16)] = jnp.where(_lane_iota() < 8, tail, _NP)

        def body(j, cnt):
            d = dbuf[pl.ds(j * 16, 16)]
            b = (d * _BMUL) >> 22
            for bb in range(_NB):
                pop = plsc.all_reduce_population_count(b == bb)
                cnt = cnt + jnp.where(_lane_iota() == bb, pop, 0)
            return cnt

        cnt = lax.fori_loop(0, _EPWP // 16, body, jnp.zeros((16,), jnp.int32))
        cvec[...] = cnt
        pltpu.sync_copy(cvec, out_hbm.at[pl.ds(w * _NB, _NB)])

    return count_kernel(dst)


def _bucket_offsets(counts):
    @functools.partial(
        pl.kernel,
        out_type=[
            jax.ShapeDtypeStruct((_NW * _NB,), jnp.int32),
            jax.ShapeDtypeStruct((16,), jnp.int32),
        ],
        mesh=plsc.ScalarSubcoreMesh(axis_name="core", num_cores=2),
        scratch_types=[
            pltpu.SMEM((_NW * _NB,), jnp.int32),
            pltpu.SMEM((_NW * _NB,), jnp.int32),
            pltpu.SMEM((16,), jnp.int32),
            pltpu.SemaphoreType.DMA,
        ],
    )
    def offs_kernel(cnt_hbm, off_hbm, tot_hbm, cbuf, obuf, tbuf, sem):
        core = lax.axis_index("core")

        @pl.when(core == 0)
        def _():
            pltpu.async_copy(cnt_hbm, cbuf, sem).wait()

            @pl.loop(0, _NB)
            def _(b):
                def inner(w, run):
                    obuf[w * _NB + b] = b * _E + run
                    return run + cbuf[w * _NB + b]

                tbuf[b] = lax.fori_loop(0, _NW, inner, 0)

            pltpu.async_copy(obuf, off_hbm, sem).wait()
            pltpu.async_copy(tbuf, tot_hbm, sem).wait()

    return offs_kernel(counts)


def _bucket_scatter(src, dst, offs):
    @functools.partial(
        pl.kernel,
        out_type=jax.ShapeDtypeStruct((_NB * _E + 16,), jnp.int32),
        mesh=_sc_mesh(),
        compiler_params=_sc_params(),
        scratch_types=[
            pltpu.VMEM((_EPWP,), jnp.int32),
            pltpu.VMEM((_EPWP,), jnp.int32),
            pltpu.VMEM((_EPWP,), jnp.int32),
            pltpu.VMEM((_EPWP,), jnp.int32),
            pltpu.VMEM((16,), jnp.int32),
            pltpu.SMEM((16,), jnp.int32),
        ],
    )
    def scat_kernel(src_hbm, dst_hbm, off_hbm, out_hbm,
                    sbuf, dbuf, pbuf, pos, off_v, osm):
        c = lax.axis_index("c")
        s = lax.axis_index("s")
        w = s * 2 + c
        pltpu.sync_copy(src_hbm.at[pl.ds(w * _EPW, _EPW)],
                        sbuf.at[pl.ds(0, _EPW)])
        pltpu.sync_copy(dst_hbm.at[pl.ds(w * _EPW, _EPW)],
                        dbuf.at[pl.ds(0, _EPW)])
        pltpu.sync_copy(off_hbm.at[pl.ds(w * _NB, _NB)], off_v)
        ov = off_v[...]
        for bb in range(_NB):
            osm[bb] = _masked_scalar(ov, bb)
        tail = dbuf[pl.ds(_EPWP - 16, 16)]
        dbuf[pl.ds(_EPWP - 16, 16)] = jnp.where(_lane_iota() < 8, tail, _NP)

        @pl.loop(0, _EPWP // 16)
        def _(j):
            sl = pl.ds(j * 16, 16)
            pbuf[sl] = sbuf[sl] | (dbuf[sl] << 14)

        @pl.loop(0, _EPWP // 16)
        def _(j):
            sl = pl.ds(j * 16, 16)
            b = (dbuf[sl] * _BMUL) >> 22
            posv = _NB * _E + _lane_iota()
            for bb in range(_NB):
                m = b == bb
                mi = m.astype(jnp.int32)
                csum = jnp.cumsum(mi)
                obb = osm[bb]
                posv = jnp.where(m, csum - 1 + obb, posv)
                osm[bb] = obb + jnp.sum(mi)
            pos[sl] = posv

        pltpu.sync_copy(pbuf, out_hbm.at[pos])

    return scat_kernel(src, dst, offs)


def _degrees(bedges, tot, zeros16):
    C = 1920

    @functools.partial(
        pl.kernel,
        out_type=jax.ShapeDtypeStruct((2, _NP, 16), jnp.float32),
        mesh=_sc_mesh(),
        compiler_params=_sc_params(),
        scratch_types=[
            pltpu.VMEM((C,), jnp.int32),
            pltpu.VMEM((C,), jnp.int32),
            pltpu.VMEM((_AR, 16), jnp.float32),
            pltpu.VMEM((16,), jnp.int32),
        ],
    )
    def deg_kernel(be_hbm, tot_hbm, z_hbm, out_hbm, ebuf, dloc, acc, totv):
        c = lax.axis_index("c")
        t = lax.axis_index("s")
        pltpu.sync_copy(z_hbm, acc)
        pltpu.sync_copy(tot_hbm, totv)
        cnt = _masked_scalar(totv[...], t)
        half = ((cnt + 1) // 2 + 7) // 8 * 8
        start = t * _E + c * half
        mycnt = jnp.where(c == 0, jnp.minimum(half, cnt),
                          jnp.maximum(cnt - half, 0))
        nch = (mycnt + (C - 1)) // C
        one = jnp.ones((16,), jnp.float32)
        cols = _lane_iota()

        def chunk(g, carry):
            pltpu.sync_copy(be_hbm.at[pl.ds(start + g * C, C)], ebuf)
            rem = mycnt - g * C

            @pl.loop(0, C // 16)
            def _(j):
                sl = pl.ds(j * 16, 16)
                v = ebuf[sl]
                m = (j * 16 + _lane_iota()) < rem
                dloc[sl] = jnp.where(m, (v >> 14) - t * _NPB, _NPB)

            def edge_body(e, cc):
                for u in range(4):
                    es = jnp.full((16,), 4 * e + u, jnp.int32)
                    row = plsc.load_gather(dloc, [es])
                    plsc.addupdate_scatter(acc, [row, cols], one)
                return cc

            lax.fori_loop(0, C // 4, edge_body, 0)
            return carry

        lax.fori_loop(0, nch, chunk, 0)
        pltpu.sync_copy(acc.at[pl.ds(0, _NPB)],
                        out_hbm.at[c, pl.ds(t * _NPB, _NPB)])

    return deg_kernel(bedges, tot, zeros16)


def _edge_aggregate(hs, bedges, tot, zeros, col_mode, dcols=_DH):
    SB = 3168
    C = 96

    @functools.partial(
        pl.kernel,
        out_type=jax.ShapeDtypeStruct((2, _NP, _DH), jnp.float32),
        mesh=_sc_mesh(),
        compiler_params=_sc_params(),
        scratch_types=[
            pltpu.VMEM((SB,), jnp.int32),
            pltpu.VMEM((SB,), jnp.int32),
            pltpu.VMEM((SB,), jnp.int32),
            pltpu.VMEM((C, _DH), jnp.float32),
            pltpu.VMEM((C, _DH), jnp.float32),
            pltpu.VMEM((_AR, _DH), jnp.float32),
            pltpu.VMEM((16,), jnp.int32),
            pltpu.SemaphoreType.DMA,
            pltpu.SemaphoreType.DMA,
        ],
    )
    def agg_kernel(hs_hbm, be_hbm, tot_hbm, z_hbm, out_hbm,
                   ebuf, sidx, dloc, buf0, buf1, acc, totv, sem0, sem1):
        c = lax.axis_index("c")
        t = lax.axis_index("s")
        pltpu.sync_copy(z_hbm, acc)
        pltpu.sync_copy(tot_hbm, totv)
        cnt = _masked_scalar(totv[...], t)
        if col_mode:
            start = t * _E
            mycnt = cnt
        else:
            half = ((cnt + 1) // 2 + 7) // 8 * 8
            start = t * _E + c * half
            mycnt = jnp.where(c == 0, jnp.minimum(half, cnt),
                              jnp.maximum(cnt - half, 0))
        nsb = (mycnt + (SB - 1)) // SB
        bufs = (buf0, buf1)
        sems = (sem0, sem1)

        def gather(g, par):
            return pltpu.make_async_copy(
                hs_hbm.at[c].at[sidx.at[pl.ds(g * C, C)]] if col_mode
                else hs_hbm.at[sidx.at[pl.ds(g * C, C)]],
                bufs[par], sems[par])

        def accumulate(base, bufP):
            def edge_body(e, cc):
                for u in range(4):
                    ee = 4 * e + u
                    es = jnp.full((16,), base + ee, jnp.int32)
                    row = plsc.load_gather(dloc, [es])
                    for cb in range(dcols // 16):
                        vals = bufP[ee, pl.ds(cb * 16, 16)]
                        plsc.addupdate_scatter(
                            acc, [row, cb * 16 + _lane_iota()], vals)
                return cc

            lax.fori_loop(0, C // 4, edge_body, 0)

        def super_chunk(sb, carry):
            sbase = start + sb * SB
            scnt = jnp.minimum(SB, mycnt - sb * SB)
            pltpu.sync_copy(be_hbm.at[pl.ds(sbase, SB)], ebuf)

            @pl.loop(0, SB // 16)
            def _(j):
                sl = pl.ds(j * 16, 16)
                v = ebuf[sl]
                m = (j * 16 + _lane_iota()) < scnt
                sidx[sl] = jnp.where(m, v & 16383, 0)
                dloc[sl] = jnp.where(m, (v >> 14) - t * _NPB, _NPB)

            nc2 = (scnt + (C - 1)) // C
            gather(0, 0).start()

            def pair(gp, cc):
                for par in (0, 1):
                    g = 2 * gp + par

                    @pl.when(g < nc2)
                    def _():
                        @pl.when(g + 1 < nc2)
                        def _():
                            gather(g + 1, 1 - par).start()

                        gather(g, par).wait()
                        accumulate(g * C, bufs[par])
                return cc

            lax.fori_loop(0, (nc2 + 1) // 2, pair, 0)
            return carry

        lax.fori_loop(0, nsb, super_chunk, 0)
        pltpu.sync_copy(acc.at[pl.ds(0, _NPB)],
                        out_hbm.at[c, pl.ds(t * _NPB, _NPB)])

    return agg_kernel(hs, bedges, tot, zeros)


def _dinv_from_deg(deg):
    def body(d_ref, o_ref):
        v = d_ref[...]
        o_ref[...] = lax.rsqrt(v[0, :, 0:1] + v[1, :, 0:1] + 1.0)

    return pl.pallas_call(
        body, out_shape=jax.ShapeDtypeStruct((_NP, 1), jnp.float32))(deg)


def _pad128(h):
    d = h.shape[1]
    if d == _DH:
        return h
    return jnp.concatenate(
        [h, jnp.zeros((h.shape[0], _DH - d), jnp.float32)], axis=-1)


def _tc_first(x, W, dinv2d):
    din, d = W.shape
    dh = d // 2

    def body(x_ref, w_ref, dinv_ref, hs_ref):
        h = jnp.dot(x_ref[...], w_ref[...],
                    preferred_element_type=jnp.float32,
                    precision=lax.Precision.HIGHEST)
        hs = h * dinv_ref[...]
        hs_ref[0] = hs[:, :dh]
        hs_ref[1] = hs[:, dh:]

    return pl.pallas_call(
        body,
        grid=(_R,),
        in_specs=[
            pl.BlockSpec((_BR, din), lambda r: (r, 0)),
            pl.BlockSpec((din, d), lambda r: (0, 0)),
            pl.BlockSpec((_BR, 1), lambda r: (r, 0)),
        ],
        out_specs=pl.BlockSpec((2, _BR, dh), lambda r: (0, r, 0)),
        out_shape=jax.ShapeDtypeStruct((2, _N, dh), jnp.float32),
    )(x, W, dinv2d)


def _tc_mid(agg, hs, b2d, dinv2d, W, in_col, skip=None, emit_act=False):
    d_prev = b2d.shape[1]
    d = W.shape[1]
    out_col = d == 256
    dh = d // 2

    def body(*refs):
        agg_r, hs_r, b_r, dinv_r, w_r = refs[:5]
        pos = 5
        skip_r = None
        if skip is not None:
            skip_r = refs[pos]
            pos += 1
        outs = refs[pos:]
        aggv = agg_r[...]
        hsv = hs_r[...]
        if in_col:
            pre = jnp.concatenate([aggv[0] + hsv[0], aggv[1] + hsv[1]],
                                  axis=-1)
        else:
            pre = (aggv[0] + aggv[1] + hsv)[:, :d_prev]
        a = dinv_r[...] * pre + b_r[...]
        if skip_r is not None:
            a = a + skip_r[...]
        act = jnp.maximum(a, 0.0)
        o = 0
        if emit_act:
            outs[o][...] = act
            o += 1
        h = jnp.dot(act, w_r[...], preferred_element_type=jnp.float32,
                    precision=lax.Precision.HIGHEST)
        hsn = h * dinv_r[...]
        if out_col:
            outs[o][0] = hsn[:, :dh]
            outs[o][1] = hsn[:, dh:]
        else:
            outs[o][...] = _pad128(hsn)

    in_specs = [
        pl.BlockSpec((2, _BR, _DH), lambda r: (0, r, 0)),
        pl.BlockSpec((2, _BR, _DH), lambda r: (0, r, 0)) if in_col
        else pl.BlockSpec((_BR, _DH), lambda r: (r, 0)),
        pl.BlockSpec((1, d_prev), lambda r: (0, 0)),
        pl.BlockSpec((_BR, 1), lambda r: (r, 0)),
        pl.BlockSpec((W.shape[0], d), lambda r: (0, 0)),
    ]
    args = [agg, hs, b2d, dinv2d, W]
    if skip is not None:
        in_specs.append(pl.BlockSpec((_BR, d_prev), lambda r: (r, 0)))
        args.append(skip)
    out_specs, out_shapes = [], []
    if emit_act:
        out_specs.append(pl.BlockSpec((_BR, d_prev), lambda r: (r, 0)))
        out_shapes.append(jax.ShapeDtypeStruct((_N, d_prev), jnp.float32))
    if out_col:
        out_specs.append(pl.BlockSpec((2, _BR, dh), lambda r: (0, r, 0)))
        out_shapes.append(jax.ShapeDtypeStruct((2, _N, dh), jnp.float32))
    else:
        out_specs.append(pl.BlockSpec((_BR, _DH), lambda r: (r, 0)))
        out_shapes.append(jax.ShapeDtypeStruct((_N, _DH), jnp.float32))

    res = pl.pallas_call(
        body, grid=(_R,), in_specs=in_specs,
        out_specs=out_specs, out_shape=out_shapes,
    )(*args)
    if emit_act:
        return res
    return res[0]


def _tc_final(agg, hs, b2d, dinv2d, Wp, bop):
    d_prev = b2d.shape[1]
    dp = Wp.shape[1]

    def body(agg_r, hs_r, b_r, dinv_r, w_r, bo_r, act_ref, z_ref):
        aggv = agg_r[...]
        pre = (aggv[0] + aggv[1] + hs_r[...])[:, :d_prev]
        act = jnp.maximum(dinv_r[...] * pre + b_r[...], 0.0)
        act_ref[...] = act
        z_ref[...] = jnp.dot(act, w_r[...],
                             preferred_element_type=jnp.float32,
                             precision=lax.Precision.HIGHEST) + bo_r[...]

    return pl.pallas_call(
        body,
        grid=(_R,),
        in_specs=[
            pl.BlockSpec((2, _BR, _DH), lambda r: (0, r, 0)),
            pl.BlockSpec((_BR, _DH), lambda r: (r, 0)),
            pl.BlockSpec((1, d_prev), lambda r: (0, 0)),
            pl.BlockSpec((_BR, 1), lambda r: (r, 0)),
            pl.BlockSpec((d_prev, dp), lambda r: (0, 0)),
            pl.BlockSpec((1, dp), lambda r: (0, 0)),
        ],
        out_specs=[
            pl.BlockSpec((_BR, d_prev), lambda r: (r, 0)),
            pl.BlockSpec((_BR, dp), lambda r: (r, 0)),
        ],
        out_shape=[
            jax.ShapeDtypeStruct((_N, d_prev), jnp.float32),
            jax.ShapeDtypeStruct((_N, dp), jnp.float32),
        ],
    )(agg, hs, b2d, dinv2d, Wp, bop)


def kernel(x, edge_index, W1, b1, W2, b2, W3, b3, W4, b4, W5, b5, W6, b6,
           W7, b7, W_out, b_out):
    src = edge_index[0]
    dst = edge_index[1]
    zeros128 = jnp.zeros((_AR, _DH), jnp.float32)
    zeros16 = jnp.zeros((_AR, 16), jnp.float32)

    counts = _bucket_counts(dst)
    offs, tot = _bucket_offsets(counts)
    bedges = _bucket_scatter(src, dst, offs)
    deg = _degrees(bedges, tot, zeros16)
    dinv2d = _dinv_from_deg(deg)

    def agg_of(hs, col_mode, dcols=_DH):
        return _edge_aggregate(hs, bedges, tot, zeros128, col_mode, dcols)

    b2ds = [b.reshape(1, -1) for b in (b1, b2, b3, b4, b5, b6, b7)]

    hs1 = _tc_first(x, W1, dinv2d)
    agg1 = agg_of(hs1, True)
    act1, hs2 = _tc_mid(agg1, hs1, b2ds[0], dinv2d, W2, in_col=True,
                        emit_act=True)
    agg2 = agg_of(hs2, True)
    hs3 = _tc_mid(agg2, hs2, b2ds[1], dinv2d, W3, in_col=True, skip=act1)
    agg3 = agg_of(hs3, False)
    act3, hs4 = _tc_mid(agg3, hs3, b2ds[2], dinv2d, W4, in_col=False,
                        emit_act=True)
    agg4 = agg_of(hs4, False)
    hs5 = _tc_mid(agg4, hs4, b2ds[3], dinv2d, W5, in_col=False, skip=act3)
    agg5 = agg_of(hs5, False, 64)
    act5, hs6 = _tc_mid(agg5, hs5, b2ds[4], dinv2d, W6, in_col=False,
                        emit_act=True)
    agg6 = agg_of(hs6, False, 64)
    hs7 = _tc_mid(agg6, hs6, b2ds[5], dinv2d, W7, in_col=False, skip=act5)
    agg7 = agg_of(hs7, False, 32)

    Wp = jnp.pad(W_out, ((0, 0), (0, 128 - W_out.shape[1])))
    bop = jnp.pad(b_out, (0, 128 - b_out.shape[0])).reshape(1, -1)
    h7, z_pad = _tc_final(agg7, hs7, b2ds[6], dinv2d, Wp, bop)
    return (h7, z_pad[:, : b_out.shape[0]])


# speedup vs baseline: 6.3220x; 1.1161x over previous
import dataclasses
import functools

import jax
import jax.numpy as jnp
from jax import lax
from jax.experimental import pallas as pl
from jax.experimental.pallas import tpu as pltpu
from jax.experimental.pallas import tpu_sc as plsc

_N = 10000
_NP = 10240
_E = 160000
_NB = 16
_NPB = _NP // _NB
_AR = 648
_NW = 32
_EPW = _E // _NW
_EPWP = 5008
_BR = 2000
_R = _N // _BR
_BMUL = 6554
_DH = 128
_C = 192


def _sc_mesh():
    return plsc.VectorSubcoreMesh(core_axis_name="c", subcore_axis_name="s")


def _sc_params():
    cp = pltpu.CompilerParams()
    if "needs_layout_passes" in pltpu.CompilerParams.__dataclass_fields__:
        cp = dataclasses.replace(cp, needs_layout_passes=False)
    return cp


def _lane_iota():
    return lax.iota(jnp.int32, 16)


def _lane_bcast(vec, k):
    idx = jnp.full((16, 1), k, jnp.int32)
    return lax.gather(
        vec, idx,
        lax.GatherDimensionNumbers(offset_dims=(), collapsed_slice_dims=(0,),
                                   start_index_map=(0,)),
        (1,), mode=lax.GatherScatterMode.PROMISE_IN_BOUNDS)


def _masked_scalar(vec16, lane):
    return jnp.sum(jnp.where(_lane_iota() == lane, vec16, 0))


def _bucket_counts(dst):
    @functools.partial(
        pl.kernel,
        out_type=jax.ShapeDtypeStruct((_NW * _NB,), jnp.int32),
        mesh=_sc_mesh(),
        compiler_params=_sc_params(),
        scratch_types=[
            pltpu.VMEM((_EPWP,), jnp.int32),
            pltpu.VMEM((16,), jnp.int32),
        ],
    )
    def count_kernel(dst_hbm, out_hbm, dbuf, cvec):
        c = lax.axis_index("c")
        s = lax.axis_index("s")
        w = s * 2 + c
        pltpu.sync_copy(dst_hbm.at[pl.ds(w * _EPW, _EPW)],
                        dbuf.at[pl.ds(0, _EPW)])
        tail = dbuf[pl.ds(_EPWP - 16, 16)]
        dbuf[pl.ds(_EPWP - 16, 16)] = jnp.where(_lane_iota() < 8, tail, _NP)

        def body(j, cnt):
            d = dbuf[pl.ds(j * 16, 16)]
            b = (d * _BMUL) >> 22
            for bb in range(_NB):
                pop = plsc.all_reduce_population_count(b == bb)
                cnt = cnt + jnp.where(_lane_iota() == bb, pop, 0)
            return cnt

        cnt = lax.fori_loop(0, _EPWP // 16, body, jnp.zeros((16,), jnp.int32))
        cvec[...] = cnt
        pltpu.sync_copy(cvec, out_hbm.at[pl.ds(w * _NB, _NB)])

    return count_kernel(dst)


def _bucket_offsets(counts):
    @functools.partial(
        pl.kernel,
        out_type=[
            jax.ShapeDtypeStruct((_NW * _NB,), jnp.int32),
            jax.ShapeDtypeStruct((16,), jnp.int32),
        ],
        mesh=plsc.ScalarSubcoreMesh(axis_name="core", num_cores=2),
        scratch_types=[
            pltpu.SMEM((_NW * _NB,), jnp.int32),
            pltpu.SMEM((_NW * _NB,), jnp.int32),
            pltpu.SMEM((16,), jnp.int32),
            pltpu.SemaphoreType.DMA,
        ],
    )
    def offs_kernel(cnt_hbm, off_hbm, tot_hbm, cbuf, obuf, tbuf, sem):
        core = lax.axis_index("core")

        @pl.when(core == 0)
        def _():
            pltpu.async_copy(cnt_hbm, cbuf, sem).wait()

            @pl.loop(0, _NB)
            def _(b):
                def inner(w, run):
                    obuf[w * _NB + b] = b * _E + run
                    return run + cbuf[w * _NB + b]

                tbuf[b] = lax.fori_loop(0, _NW, inner, 0)

            pltpu.async_copy(obuf, off_hbm, sem).wait()
            pltpu.async_copy(tbuf, tot_hbm, sem).wait()

    return offs_kernel(counts)


def _bucket_scatter(src, dst, offs):
    @functools.partial(
        pl.kernel,
        out_type=jax.ShapeDtypeStruct((_NB * _E + 16,), jnp.int32),
        mesh=_sc_mesh(),
        compiler_params=_sc_params(),
        scratch_types=[
            pltpu.VMEM((_EPWP,), jnp.int32),
            pltpu.VMEM((_EPWP,), jnp.int32),
            pltpu.VMEM((_EPWP,), jnp.int32),
            pltpu.VMEM((_EPWP,), jnp.int32),
            pltpu.VMEM((16,), jnp.int32),
            pltpu.SMEM((16,), jnp.int32),
        ],
    )
    def scat_kernel(src_hbm, dst_hbm, off_hbm, out_hbm,
                    sbuf, dbuf, pbuf, pos, off_v, osm):
        c = lax.axis_index("c")
        s = lax.axis_index("s")
        w = s * 2 + c
        pltpu.sync_copy(src_hbm.at[pl.ds(w * _EPW, _EPW)],
                        sbuf.at[pl.ds(0, _EPW)])
        pltpu.sync_copy(dst_hbm.at[pl.ds(w * _EPW, _EPW)],
                        dbuf.at[pl.ds(0, _EPW)])
        pltpu.sync_copy(off_hbm.at[pl.ds(w * _NB, _NB)], off_v)
        ov = off_v[...]
        for bb in range(_NB):
            osm[bb] = _masked_scalar(ov, bb)
        tail = dbuf[pl.ds(_EPWP - 16, 16)]
        dbuf[pl.ds(_EPWP - 16, 16)] = jnp.where(_lane_iota() < 8, tail, _NP)

        @pl.loop(0, _EPWP // 16)
        def _(j):
            sl = pl.ds(j * 16, 16)
            pbuf[sl] = sbuf[sl] | (dbuf[sl] << 14)

        @pl.loop(0, _EPWP // 16)
        def _(j):
            sl = pl.ds(j * 16, 16)
            b = (dbuf[sl] * _BMUL) >> 22
            posv = _NB * _E + _lane_iota()
            for bb in range(_NB):
                m = b == bb
                mi = m.astype(jnp.int32)
                csum = jnp.cumsum(mi)
                obb = osm[bb]
                posv = jnp.where(m, csum - 1 + obb, posv)
                osm[bb] = obb + jnp.sum(mi)
            pos[sl] = posv

        pltpu.sync_copy(pbuf, out_hbm.at[pos])

    return scat_kernel(src, dst, offs)


def _degrees(bedges, tot, zeros16):
    C = 1920

    @functools.partial(
        pl.kernel,
        out_type=jax.ShapeDtypeStruct((2, _NP, 16), jnp.float32),
        mesh=_sc_mesh(),
        compiler_params=_sc_params(),
        scratch_types=[
            pltpu.VMEM((C,), jnp.int32),
            pltpu.VMEM((C,), jnp.int32),
            pltpu.VMEM((_AR, 16), jnp.float32),
            pltpu.VMEM((16,), jnp.int32),
        ],
    )
    def deg_kernel(be_hbm, tot_hbm, z_hbm, out_hbm, ebuf, dloc, acc, totv):
        c = lax.axis_index("c")
        t = lax.axis_index("s")
        pltpu.sync_copy(z_hbm, acc)
        pltpu.sync_copy(tot_hbm, totv)
        cnt = _masked_scalar(totv[...], t)
        half = ((cnt + 1) // 2 + 7) // 8 * 8
        start = t * _E + c * half
        mycnt = jnp.where(c == 0, jnp.minimum(half, cnt),
                          jnp.maximum(cnt - half, 0))
        nch = (mycnt + (C - 1)) // C
        one = jnp.ones((16,), jnp.float32)
        cols = _lane_iota()

        def chunk(g, carry):
            pltpu.sync_copy(be_hbm.at[pl.ds(start + g * C, C)], ebuf)
            rem = mycnt - g * C

            @pl.loop(0, C // 16)
            def _(j):
                sl = pl.ds(j * 16, 16)
                v = ebuf[sl]
                m = (j * 16 + _lane_iota()) < rem
                dloc[sl] = jnp.where(m, (v >> 14) - t * _NPB, _NPB)

            def edge_body(j, cc):
                rv = dloc[pl.ds(j * 16, 16)]
                for k in range(16):
                    row = _lane_bcast(rv, k)
                    plsc.addupdate_scatter(acc, [row, cols], one)
                return cc

            lax.fori_loop(0, C // 16, edge_body, 0)
            return carry

        lax.fori_loop(0, nch, chunk, 0)
        pltpu.sync_copy(acc.at[pl.ds(0, _NPB)],
                        out_hbm.at[c, pl.ds(t * _NPB, _NPB)])

    return deg_kernel(bedges, tot, zeros16)


def _edge_aggregate(hs, bedges, tot, zeros, col_mode, dcols=_DH):
    SB = 3168
    C = 96

    @functools.partial(
        pl.kernel,
        out_type=jax.ShapeDtypeStruct((2, _NP, _DH), jnp.float32),
        mesh=_sc_mesh(),
        compiler_params=_sc_params(),
        scratch_types=[
            pltpu.VMEM((SB,), jnp.int32),
            pltpu.VMEM((SB,), jnp.int32),
            pltpu.VMEM((SB,), jnp.int32),
            pltpu.VMEM((C, _DH), jnp.float32),
            pltpu.VMEM((C, _DH), jnp.float32),
            pltpu.VMEM((_AR, _DH), jnp.float32),
            pltpu.VMEM((16,), jnp.int32),
            pltpu.SemaphoreType.DMA,
            pltpu.SemaphoreType.DMA,
        ],
    )
    def agg_kernel(hs_hbm, be_hbm, tot_hbm, z_hbm, out_hbm,
                   ebuf, sidx, dloc, buf0, buf1, acc, totv, sem0, sem1):
        c = lax.axis_index("c")
        t = lax.axis_index("s")
        pltpu.sync_copy(z_hbm, acc)
        pltpu.sync_copy(tot_hbm, totv)
        cnt = _masked_scalar(totv[...], t)
        if col_mode:
            start = t * _E
            mycnt = cnt
        else:
            half = ((cnt + 1) // 2 + 7) // 8 * 8
            start = t * _E + c * half
            mycnt = jnp.where(c == 0, jnp.minimum(half, cnt),
                              jnp.maximum(cnt - half, 0))
        nsb = (mycnt + (SB - 1)) // SB
        bufs = (buf0, buf1)
        sems = (sem0, sem1)

        def gather(g, par):
            return pltpu.make_async_copy(
                hs_hbm.at[c].at[sidx.at[pl.ds(g * C, C)]] if col_mode
                else hs_hbm.at[sidx.at[pl.ds(g * C, C)]],
                bufs[par], sems[par])

        def accumulate(base, bufP):
            def edge_body(j, cc):
                rv = dloc[pl.ds(base + j * 16, 16)]
                for k in range(16):
                    row = _lane_bcast(rv, k)
                    ee = j * 16 + k
                    for cb in range(dcols // 16):
                        vals = bufP[ee, pl.ds(cb * 16, 16)]
                        plsc.addupdate_scatter(
                            acc, [row, cb * 16 + _lane_iota()], vals)
                return cc

            lax.fori_loop(0, C // 16, edge_body, 0)

        def super_chunk(sb, carry):
            sbase = start + sb * SB
            scnt = jnp.minimum(SB, mycnt - sb * SB)
            pltpu.sync_copy(be_hbm.at[pl.ds(sbase, SB)], ebuf)

            @pl.loop(0, SB // 16)
            def _(j):
                sl = pl.ds(j * 16, 16)
                v = ebuf[sl]
                m = (j * 16 + _lane_iota()) < scnt
                sidx[sl] = jnp.where(m, v & 16383, 0)
                dloc[sl] = jnp.where(m, (v >> 14) - t * _NPB, _NPB)

            nc2 = (scnt + (C - 1)) // C
            gather(0, 0).start()

            def pair(gp, cc):
                for par in (0, 1):
                    g = 2 * gp + par

                    @pl.when(g < nc2)
                    def _():
                        @pl.when(g + 1 < nc2)
                        def _():
                            gather(g + 1, 1 - par).start()

                        gather(g, par).wait()
                        accumulate(g * C, bufs[par])
                return cc

            lax.fori_loop(0, (nc2 + 1) // 2, pair, 0)
            return carry

        lax.fori_loop(0, nsb, super_chunk, 0)
        pltpu.sync_copy(acc.at[pl.ds(0, _NPB)],
                        out_hbm.at[c, pl.ds(t * _NPB, _NPB)])

    return agg_kernel(hs, bedges, tot, zeros)


def _dinv_from_deg(deg):
    def body(d_ref, o_ref):
        v = d_ref[...]
        o_ref[...] = lax.rsqrt(v[0, :, 0:1] + v[1, :, 0:1] + 1.0)

    return pl.pallas_call(
        body, out_shape=jax.ShapeDtypeStruct((_NP, 1), jnp.float32))(deg)


def _pad128(h):
    d = h.shape[1]
    if d == _DH:
        return h
    return jnp.concatenate(
        [h, jnp.zeros((h.shape[0], _DH - d), jnp.float32)], axis=-1)


def _tc_first(x, W, dinv2d):
    din, d = W.shape
    dh = d // 2

    def body(x_ref, w_ref, dinv_ref, hs_ref):
        h = jnp.dot(x_ref[...], w_ref[...],
                    preferred_element_type=jnp.float32,
                    precision=lax.Precision.HIGHEST)
        hs = h * dinv_ref[...]
        hs_ref[0] = hs[:, :dh]
        hs_ref[1] = hs[:, dh:]

    return pl.pallas_call(
        body,
        grid=(_R,),
        in_specs=[
            pl.BlockSpec((_BR, din), lambda r: (r, 0)),
            pl.BlockSpec((din, d), lambda r: (0, 0)),
            pl.BlockSpec((_BR, 1), lambda r: (r, 0)),
        ],
        out_specs=pl.BlockSpec((2, _BR, dh), lambda r: (0, r, 0)),
        out_shape=jax.ShapeDtypeStruct((2, _N, dh), jnp.float32),
    )(x, W, dinv2d)


def _tc_mid(agg, hs, b2d, dinv2d, W, in_col, skip=None, emit_act=False):
    d_prev = b2d.shape[1]
    d = W.shape[1]
    out_col = d == 256
    dh = d // 2

    def body(*refs):
        agg_r, hs_r, b_r, dinv_r, w_r = refs[:5]
        pos = 5
        skip_r = None
        if skip is not None:
            skip_r = refs[pos]
            pos += 1
        outs = refs[pos:]
        aggv = agg_r[...]
        hsv = hs_r[...]
        if in_col:
            pre = jnp.concatenate([aggv[0] + hsv[0], aggv[1] + hsv[1]],
                                  axis=-1)
        else:
            pre = (aggv[0] + aggv[1] + hsv)[:, :d_prev]
        a = dinv_r[...] * pre + b_r[...]
        if skip_r is not None:
            a = a + skip_r[...]
        act = jnp.maximum(a, 0.0)
        o = 0
        if emit_act:
            outs[o][...] = act
            o += 1
        h = jnp.dot(act, w_r[...], preferred_element_type=jnp.float32,
                    precision=lax.Precision.HIGHEST)
        hsn = h * dinv_r[...]
        if out_col:
            outs[o][0] = hsn[:, :dh]
            outs[o][1] = hsn[:, dh:]
        else:
            outs[o][...] = _pad128(hsn)

    in_specs = [
        pl.BlockSpec((2, _BR, _DH), lambda r: (0, r, 0)),
        pl.BlockSpec((2, _BR, _DH), lambda r: (0, r, 0)) if in_col
        else pl.BlockSpec((_BR, _DH), lambda r: (r, 0)),
        pl.BlockSpec((1, d_prev), lambda r: (0, 0)),
        pl.BlockSpec((_BR, 1), lambda r: (r, 0)),
        pl.BlockSpec((W.shape[0], d), lambda r: (0, 0)),
    ]
    args = [agg, hs, b2d, dinv2d, W]
    if skip is not None:
        in_specs.append(pl.BlockSpec((_BR, d_prev), lambda r: (r, 0)))
        args.append(skip)
    out_specs, out_shapes = [], []
    if emit_act:
        out_specs.append(pl.BlockSpec((_BR, d_prev), lambda r: (r, 0)))
        out_shapes.append(jax.ShapeDtypeStruct((_N, d_prev), jnp.float32))
    if out_col:
        out_specs.append(pl.BlockSpec((2, _BR, dh), lambda r: (0, r, 0)))
        out_shapes.append(jax.ShapeDtypeStruct((2, _N, dh), jnp.float32))
    else:
        out_specs.append(pl.BlockSpec((_BR, _DH), lambda r: (r, 0)))
        out_shapes.append(jax.ShapeDtypeStruct((_N, _DH), jnp.float32))

    res = pl.pallas_call(
        body, grid=(_R,), in_specs=in_specs,
        out_specs=out_specs, out_shape=out_shapes,
    )(*args)
    if emit_act:
        return res
    return res[0]


def _tc_final(agg, hs, b2d, dinv2d, Wp, bop):
    d_prev = b2d.shape[1]
    dp = Wp.shape[1]

    def body(agg_r, hs_r, b_r, dinv_r, w_r, bo_r, act_ref, z_ref):
        aggv = agg_r[...]
        pre = (aggv[0] + aggv[1] + hs_r[...])[:, :d_prev]
        act = jnp.maximum(dinv_r[...] * pre + b_r[...], 0.0)
        act_ref[...] = act
        z_ref[...] = jnp.dot(act, w_r[...],
                             preferred_element_type=jnp.float32,
                             precision=lax.Precision.HIGHEST) + bo_r[...]

    return pl.pallas_call(
        body,
        grid=(_R,),
        in_specs=[
            pl.BlockSpec((2, _BR, _DH), lambda r: (0, r, 0)),
            pl.BlockSpec((_BR, _DH), lambda r: (r, 0)),
            pl.BlockSpec((1, d_prev), lambda r: (0, 0)),
            pl.BlockSpec((_BR, 1), lambda r: (r, 0)),
            pl.BlockSpec((d_prev, dp), lambda r: (0, 0)),
            pl.BlockSpec((1, dp), lambda r: (0, 0)),
        ],
        out_specs=[
            pl.BlockSpec((_BR, d_prev), lambda r: (r, 0)),
            pl.BlockSpec((_BR, dp), lambda r: (r, 0)),
        ],
        out_shape=[
            jax.ShapeDtypeStruct((_N, d_prev), jnp.float32),
            jax.ShapeDtypeStruct((_N, dp), jnp.float32),
        ],
    )(agg, hs, b2d, dinv2d, Wp, bop)


def kernel(x, edge_index, W1, b1, W2, b2, W3, b3, W4, b4, W5, b5, W6, b6,
           W7, b7, W_out, b_out):
    src = edge_index[0]
    dst = edge_index[1]
    zeros128 = jnp.zeros((_AR, _DH), jnp.float32)
    zeros16 = jnp.zeros((_AR, 16), jnp.float32)

    counts = _bucket_counts(dst)
    offs, tot = _bucket_offsets(counts)
    bedges = _bucket_scatter(src, dst, offs)
    deg = _degrees(bedges, tot, zeros16)
    dinv2d = _dinv_from_deg(deg)

    def agg_of(hs, col_mode, dcols=_DH):
        return _edge_aggregate(hs, bedges, tot, zeros128, col_mode, dcols)

    b2ds = [b.reshape(1, -1) for b in (b1, b2, b3, b4, b5, b6, b7)]

    hs1 = _tc_first(x, W1, dinv2d)
    agg1 = agg_of(hs1, True)
    act1, hs2 = _tc_mid(agg1, hs1, b2ds[0], dinv2d, W2, in_col=True,
                        emit_act=True)
    agg2 = agg_of(hs2, True)
    hs3 = _tc_mid(agg2, hs2, b2ds[1], dinv2d, W3, in_col=True, skip=act1)
    agg3 = agg_of(hs3, False)
    act3, hs4 = _tc_mid(agg3, hs3, b2ds[2], dinv2d, W4, in_col=False,
                        emit_act=True)
    agg4 = agg_of(hs4, False)
    hs5 = _tc_mid(agg4, hs4, b2ds[3], dinv2d, W5, in_col=False, skip=act3)
    agg5 = agg_of(hs5, False, 64)
    act5, hs6 = _tc_mid(agg5, hs5, b2ds[4], dinv2d, W6, in_col=False,
                        emit_act=True)
    agg6 = agg_of(hs6, False, 64)
    hs7 = _tc_mid(agg6, hs6, b2ds[5], dinv2d, W7, in_col=False, skip=act5)
    agg7 = agg_of(hs7, False, 32)

    Wp = jnp.pad(W_out, ((0, 0), (0, 128 - W_out.shape[1])))
    bop = jnp.pad(b_out, (0, 128 - b_out.shape[0])).reshape(1, -1)
    h7, z_pad = _tc_final(agg7, hs7, b2ds[6], dinv2d, Wp, bop)
    return (h7, z_pad[:, : b_out.shape[0]])


# speedup vs baseline: 6.3937x; 1.0113x over previous
import dataclasses
import functools

import jax
import jax.numpy as jnp
from jax import lax
from jax.experimental import pallas as pl
from jax.experimental.pallas import tpu as pltpu
from jax.experimental.pallas import tpu_sc as plsc

_N = 10000
_NP = 10240
_E = 160000
_NB = 16
_NPB = _NP // _NB
_AR = 648
_NW = 32
_EPW = _E // _NW
_EPWP = 5008
_BR = 2000
_R = _N // _BR
_BMUL = 6554
_DH = 128
_C = 192


def _sc_mesh():
    return plsc.VectorSubcoreMesh(core_axis_name="c", subcore_axis_name="s")


def _sc_params():
    cp = pltpu.CompilerParams()
    if "needs_layout_passes" in pltpu.CompilerParams.__dataclass_fields__:
        cp = dataclasses.replace(cp, needs_layout_passes=False)
    return cp


def _lane_iota():
    return lax.iota(jnp.int32, 16)


def _lane_bcast(vec, k):
    idx = jnp.full((16, 1), k, jnp.int32)
    return lax.gather(
        vec, idx,
        lax.GatherDimensionNumbers(offset_dims=(), collapsed_slice_dims=(0,),
                                   start_index_map=(0,)),
        (1,), mode=lax.GatherScatterMode.PROMISE_IN_BOUNDS)


def _masked_scalar(vec16, lane):
    return jnp.sum(jnp.where(_lane_iota() == lane, vec16, 0))


def _bucket_counts(dst):
    @functools.partial(
        pl.kernel,
        out_type=jax.ShapeDtypeStruct((_NW * _NB,), jnp.int32),
        mesh=_sc_mesh(),
        compiler_params=_sc_params(),
        scratch_types=[
            pltpu.VMEM((_EPWP,), jnp.int32),
            pltpu.VMEM((16,), jnp.int32),
        ],
    )
    def count_kernel(dst_hbm, out_hbm, dbuf, cvec):
        c = lax.axis_index("c")
        s = lax.axis_index("s")
        w = s * 2 + c
        pltpu.sync_copy(dst_hbm.at[pl.ds(w * _EPW, _EPW)],
                        dbuf.at[pl.ds(0, _EPW)])
        tail = dbuf[pl.ds(_EPWP - 16, 16)]
        dbuf[pl.ds(_EPWP - 16, 16)] = jnp.where(_lane_iota() < 8, tail, _NP)

        def body(j, cnt):
            d = dbuf[pl.ds(j * 16, 16)]
            b = (d * _BMUL) >> 22
            for bb in range(_NB):
                pop = plsc.all_reduce_population_count(b == bb)
                cnt = cnt + jnp.where(_lane_iota() == bb, pop, 0)
            return cnt

        cnt = lax.fori_loop(0, _EPWP // 16, body, jnp.zeros((16,), jnp.int32))
        cvec[...] = cnt
        pltpu.sync_copy(cvec, out_hbm.at[pl.ds(w * _NB, _NB)])

    return count_kernel(dst)


def _bucket_offsets(counts):
    @functools.partial(
        pl.kernel,
        out_type=[
            jax.ShapeDtypeStruct((_NW * _NB,), jnp.int32),
            jax.ShapeDtypeStruct((16,), jnp.int32),
        ],
        mesh=plsc.ScalarSubcoreMesh(axis_name="core", num_cores=2),
        scratch_types=[
            pltpu.SMEM((_NW * _NB,), jnp.int32),
            pltpu.SMEM((_NW * _NB,), jnp.int32),
            pltpu.SMEM((16,), jnp.int32),
            pltpu.SemaphoreType.DMA,
        ],
    )
    def offs_kernel(cnt_hbm, off_hbm, tot_hbm, cbuf, obuf, tbuf, sem):
        core = lax.axis_index("core")

        @pl.when(core == 0)
        def _():
            pltpu.async_copy(cnt_hbm, cbuf, sem).wait()

            @pl.loop(0, _NB)
            def _(b):
                def inner(w, run):
                    obuf[w * _NB + b] = b * _E + run
                    return run + cbuf[w * _NB + b]

                tbuf[b] = lax.fori_loop(0, _NW, inner, 0)

            pltpu.async_copy(obuf, off_hbm, sem).wait()
            pltpu.async_copy(tbuf, tot_hbm, sem).wait()

    return offs_kernel(counts)


def _bucket_scatter(src, dst, offs):
    @functools.partial(
        pl.kernel,
        out_type=jax.ShapeDtypeStruct((_NB * _E + 16,), jnp.int32),
        mesh=_sc_mesh(),
        compiler_params=_sc_params(),
        scratch_types=[
            pltpu.VMEM((_EPWP,), jnp.int32),
            pltpu.VMEM((_EPWP,), jnp.int32),
            pltpu.VMEM((_EPWP,), jnp.int32),
            pltpu.VMEM((_EPWP,), jnp.int32),
            pltpu.VMEM((16,), jnp.int32),
            pltpu.SMEM((16,), jnp.int32),
        ],
    )
    def scat_kernel(src_hbm, dst_hbm, off_hbm, out_hbm,
                    sbuf, dbuf, pbuf, pos, off_v, osm):
        c = lax.axis_index("c")
        s = lax.axis_index("s")
        w = s * 2 + c
        pltpu.sync_copy(src_hbm.at[pl.ds(w * _EPW, _EPW)],
                        sbuf.at[pl.ds(0, _EPW)])
        pltpu.sync_copy(dst_hbm.at[pl.ds(w * _EPW, _EPW)],
                        dbuf.at[pl.ds(0, _EPW)])
        pltpu.sync_copy(off_hbm.at[pl.ds(w * _NB, _NB)], off_v)
        ov = off_v[...]
        for bb in range(_NB):
            osm[bb] = _masked_scalar(ov, bb)
        tail = dbuf[pl.ds(_EPWP - 16, 16)]
        dbuf[pl.ds(_EPWP - 16, 16)] = jnp.where(_lane_iota() < 8, tail, _NP)

        @pl.loop(0, _EPWP // 16)
        def _(j):
            sl = pl.ds(j * 16, 16)
            pbuf[sl] = sbuf[sl] | (dbuf[sl] << 14)

        @pl.loop(0, _EPWP // 16)
        def _(j):
            sl = pl.ds(j * 16, 16)
            b = (dbuf[sl] * _BMUL) >> 22
            posv = _NB * _E + _lane_iota()
            for bb in range(_NB):
                m = b == bb
                mi = m.astype(jnp.int32)
                csum = jnp.cumsum(mi)
                obb = osm[bb]
                posv = jnp.where(m, csum - 1 + obb, posv)
                osm[bb] = obb + jnp.sum(mi)
            pos[sl] = posv

        pltpu.sync_copy(pbuf, out_hbm.at[pos])

    return scat_kernel(src, dst, offs)


def _degrees(bedges, tot, zeros16):
    C = 1920

    @functools.partial(
        pl.kernel,
        out_type=jax.ShapeDtypeStruct((2, _NP, 16), jnp.float32),
        mesh=_sc_mesh(),
        compiler_params=_sc_params(),
        scratch_types=[
            pltpu.VMEM((C,), jnp.int32),
            pltpu.VMEM((C,), jnp.int32),
            pltpu.VMEM((_AR, 16), jnp.float32),
            pltpu.VMEM((16,), jnp.int32),
        ],
    )
    def deg_kernel(be_hbm, tot_hbm, z_hbm, out_hbm, ebuf, dloc, acc, totv):
        c = lax.axis_index("c")
        t = lax.axis_index("s")
        pltpu.sync_copy(z_hbm, acc)
        pltpu.sync_copy(tot_hbm, totv)
        cnt = _masked_scalar(totv[...], t)
        half = ((cnt + 1) // 2 + 7) // 8 * 8
        start = t * _E + c * half
        mycnt = jnp.where(c == 0, jnp.minimum(half, cnt),
                          jnp.maximum(cnt - half, 0))
        nch = (mycnt + (C - 1)) // C
        one = jnp.ones((16,), jnp.float32)
        cols = _lane_iota()

        def chunk(g, carry):
            pltpu.sync_copy(be_hbm.at[pl.ds(start + g * C, C)], ebuf)
            rem = mycnt - g * C

            @pl.loop(0, C // 16)
            def _(j):
                sl = pl.ds(j * 16, 16)
                v = ebuf[sl]
                m = (j * 16 + _lane_iota()) < rem
                dloc[sl] = jnp.where(m, (v >> 14) - t * _NPB, _NPB)

            def edge_body(j, cc):
                rv = dloc[pl.ds(j * 16, 16)]
                for k in range(16):
                    row = _lane_bcast(rv, k)
                    plsc.addupdate_scatter(acc, [row, cols], one)
                return cc

            lax.fori_loop(0, C // 16, edge_body, 0)
            return carry

        lax.fori_loop(0, nch, chunk, 0)
        pltpu.sync_copy(acc.at[pl.ds(0, _NPB)],
                        out_hbm.at[c, pl.ds(t * _NPB, _NPB)])

    return deg_kernel(bedges, tot, zeros16)


def _edge_aggregate(hs, bedges, tot, zeros, col_mode, dcols=_DH):
    SB = 3168
    C = 96

    @functools.partial(
        pl.kernel,
        out_type=jax.ShapeDtypeStruct((2, _NP, _DH), jnp.float32),
        mesh=_sc_mesh(),
        compiler_params=_sc_params(),
        scratch_types=[
            pltpu.VMEM((SB,), jnp.int32),
            pltpu.VMEM((SB,), jnp.int32),
            pltpu.VMEM((SB,), jnp.int32),
            pltpu.VMEM((C, _DH), jnp.float32),
            pltpu.VMEM((C, _DH), jnp.float32),
            pltpu.VMEM((_AR, _DH), jnp.float32),
            pltpu.VMEM((16,), jnp.int32),
            pltpu.SemaphoreType.DMA,
            pltpu.SemaphoreType.DMA,
        ],
    )
    def agg_kernel(hs_hbm, be_hbm, tot_hbm, z_hbm, out_hbm,
                   ebuf, sidx, dloc, buf0, buf1, acc, totv, sem0, sem1):
        c = lax.axis_index("c")
        t = lax.axis_index("s")
        pltpu.sync_copy(z_hbm, acc)
        pltpu.sync_copy(tot_hbm, totv)
        cnt = _masked_scalar(totv[...], t)
        if col_mode:
            start = t * _E
            mycnt = cnt
        else:
            half = ((cnt + 1) // 2 + 7) // 8 * 8
            start = t * _E + c * half
            mycnt = jnp.where(c == 0, jnp.minimum(half, cnt),
                              jnp.maximum(cnt - half, 0))
        nsb = (mycnt + (SB - 1)) // SB
        bufs = (buf0, buf1)
        sems = (sem0, sem1)

        def gather(g, par):
            return pltpu.make_async_copy(
                hs_hbm.at[c].at[sidx.at[pl.ds(g * C, C)]] if col_mode
                else hs_hbm.at[sidx.at[pl.ds(g * C, C)]],
                bufs[par], sems[par])

        def accumulate(base, bufP):
            def edge_body(j, cc):
                rv = dloc[pl.ds(base + j * 16, 16)]
                for k in range(16):
                    row = _lane_bcast(rv, k)
                    ee = j * 16 + k
                    for cb in range(dcols // 16):
                        vals = bufP[ee, pl.ds(cb * 16, 16)]
                        plsc.addupdate_scatter(
                            acc, [row, cb * 16 + _lane_iota()], vals)
                return cc

            lax.fori_loop(0, C // 16, edge_body, 0)

        def super_chunk(sb, carry):
            sbase = start + sb * SB
            scnt = jnp.minimum(SB, mycnt - sb * SB)
            pltpu.sync_copy(be_hbm.at[pl.ds(sbase, SB)], ebuf)

            @pl.loop(0, SB // 16)
            def _(j):
                sl = pl.ds(j * 16, 16)
                v = ebuf[sl]
                m = (j * 16 + _lane_iota()) < scnt
                sidx[sl] = jnp.where(m, v & 16383, 0)
                dloc[sl] = jnp.where(m, (v >> 14) - t * _NPB, _NPB)

            nc2 = (scnt + (C - 1)) // C
            gather(0, 0).start()

            def pair(gp, cc):
                for par in (0, 1):
                    g = 2 * gp + par

                    @pl.when(g < nc2)
                    def _():
                        @pl.when(g + 1 < nc2)
                        def _():
                            gather(g + 1, 1 - par).start()

                        gather(g, par).wait()
                        accumulate(g * C, bufs[par])
                return cc

            lax.fori_loop(0, (nc2 + 1) // 2, pair, 0)
            return carry

        lax.fori_loop(0, nsb, super_chunk, 0)
        pltpu.sync_copy(acc.at[pl.ds(0, _NPB)],
                        out_hbm.at[c, pl.ds(t * _NPB, _NPB)])

    return agg_kernel(hs, bedges, tot, zeros)


def _dinv_from_deg(deg):
    def body(d_ref, o_ref):
        v = d_ref[...]
        x = v[0, :, 0:1] + v[1, :, 0:1] + 1.0
        y = lax.rsqrt(x)
        o_ref[...] = y * (1.5 - 0.5 * x * y * y)

    return pl.pallas_call(
        body, out_shape=jax.ShapeDtypeStruct((_NP, 1), jnp.float32))(deg)


def _pad128(h):
    d = h.shape[1]
    if d == _DH:
        return h
    return jnp.concatenate(
        [h, jnp.zeros((h.shape[0], _DH - d), jnp.float32)], axis=-1)


def _tc_first(x, W, dinv2d):
    din, d = W.shape
    dh = d // 2

    def body(x_ref, w_ref, dinv_ref, hs_ref):
        h = jnp.dot(x_ref[...], w_ref[...],
                    preferred_element_type=jnp.float32)
        hs = h * dinv_ref[...]
        hs_ref[0] = hs[:, :dh]
        hs_ref[1] = hs[:, dh:]

    return pl.pallas_call(
        body,
        grid=(_R,),
        in_specs=[
            pl.BlockSpec((_BR, din), lambda r: (r, 0)),
            pl.BlockSpec((din, d), lambda r: (0, 0)),
            pl.BlockSpec((_BR, 1), lambda r: (r, 0)),
        ],
        out_specs=pl.BlockSpec((2, _BR, dh), lambda r: (0, r, 0)),
        out_shape=jax.ShapeDtypeStruct((2, _N, dh), jnp.float32),
    )(x, W, dinv2d)


def _tc_mid(agg, hs, b2d, dinv2d, W, in_col, skip=None, emit_act=False):
    d_prev = b2d.shape[1]
    d = W.shape[1]
    out_col = d == 256
    dh = d // 2

    def body(*refs):
        agg_r, hs_r, b_r, dinv_r, w_r = refs[:5]
        pos = 5
        skip_r = None
        if skip is not None:
            skip_r = refs[pos]
            pos += 1
        outs = refs[pos:]
        aggv = agg_r[...]
        hsv = hs_r[...]
        if in_col:
            pre = jnp.concatenate([aggv[0] + hsv[0], aggv[1] + hsv[1]],
                                  axis=-1)
        else:
            pre = (aggv[0] + aggv[1] + hsv)[:, :d_prev]
        a = dinv_r[...] * pre + b_r[...]
        if skip_r is not None:
            a = a + skip_r[...]
        act = jnp.maximum(a, 0.0)
        o = 0
        if emit_act:
            outs[o][...] = act
            o += 1
        h = jnp.dot(act, w_r[...], preferred_element_type=jnp.float32)
        hsn = h * dinv_r[...]
        if out_col:
            outs[o][0] = hsn[:, :dh]
            outs[o][1] = hsn[:, dh:]
        else:
            outs[o][...] = _pad128(hsn)

    in_specs = [
        pl.BlockSpec((2, _BR, _DH), lambda r: (0, r, 0)),
        pl.BlockSpec((2, _BR, _DH), lambda r: (0, r, 0)) if in_col
        else pl.BlockSpec((_BR, _DH), lambda r: (r, 0)),
        pl.BlockSpec((1, d_prev), lambda r: (0, 0)),
        pl.BlockSpec((_BR, 1), lambda r: (r, 0)),
        pl.BlockSpec((W.shape[0], d), lambda r: (0, 0)),
    ]
    args = [agg, hs, b2d, dinv2d, W]
    if skip is not None:
        in_specs.append(pl.BlockSpec((_BR, d_prev), lambda r: (r, 0)))
        args.append(skip)
    out_specs, out_shapes = [], []
    if emit_act:
        out_specs.append(pl.BlockSpec((_BR, d_prev), lambda r: (r, 0)))
        out_shapes.append(jax.ShapeDtypeStruct((_N, d_prev), jnp.float32))
    if out_col:
        out_specs.append(pl.BlockSpec((2, _BR, dh), lambda r: (0, r, 0)))
        out_shapes.append(jax.ShapeDtypeStruct((2, _N, dh), jnp.float32))
    else:
        out_specs.append(pl.BlockSpec((_BR, _DH), lambda r: (r, 0)))
        out_shapes.append(jax.ShapeDtypeStruct((_N, _DH), jnp.float32))

    res = pl.pallas_call(
        body, grid=(_R,), in_specs=in_specs,
        out_specs=out_specs, out_shape=out_shapes,
    )(*args)
    if emit_act:
        return res
    return res[0]


def _tc_final(agg, hs, b2d, dinv2d, Wp, bop):
    d_prev = b2d.shape[1]
    dp = Wp.shape[1]

    def body(agg_r, hs_r, b_r, dinv_r, w_r, bo_r, act_ref, z_ref):
        aggv = agg_r[...]
        pre = (aggv[0] + aggv[1] + hs_r[...])[:, :d_prev]
        act = jnp.maximum(dinv_r[...] * pre + b_r[...], 0.0)
        act_ref[...] = act
        z_ref[...] = jnp.dot(act, w_r[...],
                             preferred_element_type=jnp.float32) + bo_r[...]

    return pl.pallas_call(
        body,
        grid=(_R,),
        in_specs=[
            pl.BlockSpec((2, _BR, _DH), lambda r: (0, r, 0)),
            pl.BlockSpec((_BR, _DH), lambda r: (r, 0)),
            pl.BlockSpec((1, d_prev), lambda r: (0, 0)),
            pl.BlockSpec((_BR, 1), lambda r: (r, 0)),
            pl.BlockSpec((d_prev, dp), lambda r: (0, 0)),
            pl.BlockSpec((1, dp), lambda r: (0, 0)),
        ],
        out_specs=[
            pl.BlockSpec((_BR, d_prev), lambda r: (r, 0)),
            pl.BlockSpec((_BR, dp), lambda r: (r, 0)),
        ],
        out_shape=[
            jax.ShapeDtypeStruct((_N, d_prev), jnp.float32),
            jax.ShapeDtypeStruct((_N, dp), jnp.float32),
        ],
    )(agg, hs, b2d, dinv2d, Wp, bop)


def kernel(x, edge_index, W1, b1, W2, b2, W3, b3, W4, b4, W5, b5, W6, b6,
           W7, b7, W_out, b_out):
    src = edge_index[0]
    dst = edge_index[1]
    zeros128 = jnp.zeros((_AR, _DH), jnp.float32)
    zeros16 = jnp.zeros((_AR, 16), jnp.float32)

    counts = _bucket_counts(dst)
    offs, tot = _bucket_offsets(counts)
    bedges = _bucket_scatter(src, dst, offs)
    deg = _degrees(bedges, tot, zeros16)
    dinv2d = _dinv_from_deg(deg)

    def agg_of(hs, col_mode, dcols=_DH):
        return _edge_aggregate(hs, bedges, tot, zeros128, col_mode, dcols)

    b2ds = [b.reshape(1, -1) for b in (b1, b2, b3, b4, b5, b6, b7)]

    hs1 = _tc_first(x, W1, dinv2d)
    agg1 = agg_of(hs1, True)
    act1, hs2 = _tc_mid(agg1, hs1, b2ds[0], dinv2d, W2, in_col=True,
                        emit_act=True)
    agg2 = agg_of(hs2, True)
    hs3 = _tc_mid(agg2, hs2, b2ds[1], dinv2d, W3, in_col=True, skip=act1)
    agg3 = agg_of(hs3, False)
    act3, hs4 = _tc_mid(agg3, hs3, b2ds[2], dinv2d, W4, in_col=False,
                        emit_act=True)
    agg4 = agg_of(hs4, False)
    hs5 = _tc_mid(agg4, hs4, b2ds[3], dinv2d, W5, in_col=False, skip=act3)
    agg5 = agg_of(hs5, False, 64)
    act5, hs6 = _tc_mid(agg5, hs5, b2ds[4], dinv2d, W6, in_col=False,
                        emit_act=True)
    agg6 = agg_of(hs6, False, 64)
    hs7 = _tc_mid(agg6, hs6, b2ds[5], dinv2d, W7, in_col=False, skip=act5)
    agg7 = agg_of(hs7, False, 32)

    Wp = jnp.pad(W_out, ((0, 0), (0, 128 - W_out.shape[1])))
    bop = jnp.pad(b_out, (0, 128 - b_out.shape[0])).reshape(1, -1)
    h7, z_pad = _tc_final(agg7, hs7, b2ds[6], dinv2d, Wp, bop)
    return (h7, z_pad[:, : b_out.shape[0]])
